# baseline (speedup 1.0000x reference)
import functools

import jax
import jax.numpy as jnp
from jax import lax
from jax.experimental import pallas as pl
from jax.experimental.pallas import tpu as pltpu
from jax.experimental.pallas import tpu_sc as plsc

N = 10000
D = 128
H = 128
K = 4
NLAYER = 2
OUT = 10
ALPHA = 1.0
PA = 1.0
PB = 1.0

NPAD = 10240
E2 = 2 * 320000
NW = 32
CHUNK = 112
IB = 16
CPW = -(-E2 // (NW * CHUNK * IB)) * IB
EPAD = NW * CPW * CHUNK
TRASH = NPAD - 1
ROWS_PW = NPAD // 16

BR = 1024
GRID = NPAD // BR

f32 = jnp.float32
i32 = jnp.int32


def _shift1(v):
    return jnp.concatenate([jnp.zeros((1,), v.dtype), v[:-1]])


def _expansion(prev, L, al, a, b, l=-1.0, r=1.0):
    if L == 0:
        return prev[0]
    if L == 1:
        coef1 = ((a - b) / 2.0 - (a + b + 2.0) / 2.0 * (l + r) / (r - l)) * al[0]
        coef2 = ((a + b + 2.0) / (r - l)) * al[0]
        t1 = prev[-1]
        return coef1 * t1 + coef2 * _shift1(t1)
    coef_l = 2.0 * L * (L + a + b) * (2.0 * L - 2.0 + a + b)
    coef_lm1_1 = (2.0 * L + a + b - 1.0) * (2.0 * L + a + b) * (2.0 * L + a + b - 2.0)
    coef_lm1_2 = (2.0 * L + a + b - 1.0) * (a ** 2 - b ** 2)
    coef_lm2 = 2.0 * (L - 1.0 + a) * (L - 1.0 + b) * (2.0 * L + a + b)
    tmp1 = al[L - 1] * (coef_lm1_1 / coef_l)
    tmp2 = al[L - 1] * (coef_lm1_2 / coef_l)
    tmp3 = al[L - 1] * al[L - 2] * (coef_lm2 / coef_l)
    tmp1_2 = tmp1 * (2.0 / (r - l))
    tmp2_2 = tmp1 * ((r + l) / (r - l)) + tmp2
    t1 = prev[-1]
    return tmp1_2 * _shift1(t1) - tmp2_2 * t1 - tmp3 * prev[-2]


def _satt(att_i, al):
    tmp0 = jnp.zeros((K + 1,), f32).at[0].set(1.0)
    xs = [tmp0]
    s = att_i[:, 0][:, None] * tmp0[None, :]
    for j in range(1, K + 1):
        tx = _expansion(xs, j, al, PA, PB)
        s = s + att_i[:, j][:, None] * tx[None, :]
        xs.append(tx)
    return s.T


_MESH = plsc.VectorSubcoreMesh(core_axis_name="c", subcore_axis_name="s")


@functools.partial(
    pl.kernel,
    mesh=_MESH,
    out_type=[jax.ShapeDtypeStruct((NPAD, 16), f32),
              jax.ShapeDtypeStruct((NPAD, 16), f32)],
    scratch_types=[
        pltpu.VMEM_SHARED((NPAD, 16), f32),
        pltpu.VMEM((IB, CHUNK), i32),
        pltpu.VMEM((CHUNK, 16), f32),
        pltpu.VMEM((CHUNK, 16), f32),
    ],
)
def _deg_k(dst_hbm, degA_hbm, degB_hbm, acc_sh, dst_v, ones_v, zeros_v):
    cid = lax.axis_index("c")
    sid = lax.axis_index("s")
    wid = sid * 2 + cid
    base = sid * ROWS_PW

    def fill(rr, _):
        ones_v[rr] = jnp.full((16,), 1.0, f32)
        zeros_v[rr] = jnp.zeros((16,), f32)
        return 0

    lax.fori_loop(0, CHUNK, fill, 0)
    nz = ROWS_PW // CHUNK
    rem = ROWS_PW - nz * CHUNK
    for b in range(nz):
        pltpu.sync_copy(zeros_v, acc_sh.at[pl.ds(base + b * CHUNK, CHUNK)])
    if rem:
        pltpu.sync_copy(zeros_v.at[pl.ds(0, rem)],
                        acc_sh.at[pl.ds(base + nz * CHUNK, rem)])
    plsc.subcore_barrier()

    def blk_body(bb, _):
        pltpu.sync_copy(dst_hbm.at[wid].at[pl.ds(bb * IB, IB)], dst_v)

        def chunk_body(c, _):
            pltpu.sync_copy(ones_v, acc_sh.at[dst_v.at[c]], add=True)
            return 0

        lax.fori_loop(0, IB, chunk_body, 0)
        return 0

    lax.fori_loop(0, CPW // IB, blk_body, 0)
    plsc.subcore_barrier()

    @pl.when(cid == 0)
    def _():
        pltpu.sync_copy(acc_sh.at[pl.ds(base, ROWS_PW)],
                        degA_hbm.at[pl.ds(base, ROWS_PW)])

    @pl.when(cid == 1)
    def _():
        pltpu.sync_copy(acc_sh.at[pl.ds(base, ROWS_PW)],
                        degB_hbm.at[pl.ds(base, ROWS_PW)])


@functools.partial(
    pl.kernel,
    mesh=_MESH,
    out_type=[jax.ShapeDtypeStruct((NPAD, H), f32),
              jax.ShapeDtypeStruct((NPAD, H), f32)],
    scratch_types=[
        pltpu.VMEM_SHARED((NPAD, H), f32),
        pltpu.VMEM((IB, CHUNK), i32),
        pltpu.VMEM((IB, CHUNK), i32),
        pltpu.VMEM((CHUNK, H), f32),
        pltpu.VMEM((CHUNK, H), f32),
        pltpu.VMEM((CHUNK, H), f32),
        pltpu.SemaphoreType.DMA,
        pltpu.SemaphoreType.DMA,
        pltpu.SemaphoreType.DMA,
        pltpu.SemaphoreType.DMA,
        pltpu.SemaphoreType.DMA,
        pltpu.SemaphoreType.DMA,
    ],
)
def _prop_k(u_hbm, src_hbm, dst_hbm, accA_hbm, accB_hbm,
            acc_sh, src_v, dst_v, rows0_v, rows1_v, rows2_v,
            sg0, sg1, sg2, ss0, ss1, ss2):
    cid = lax.axis_index("c")
    sid = lax.axis_index("s")
    wid = sid * 2 + cid
    base = sid * ROWS_PW
    zeros16 = jnp.zeros((16,), f32)
    rows = [rows0_v, rows1_v, rows2_v]
    sgs = [sg0, sg1, sg2]
    sss = [ss0, ss1, ss2]

    def zrow(rr, _):
        for k in range(H // 16):
            rows0_v[rr, k * 16:(k + 1) * 16] = zeros16
        return 0

    lax.fori_loop(0, CHUNK, zrow, 0)
    nz = ROWS_PW // CHUNK
    rem = ROWS_PW - nz * CHUNK
    for b in range(nz):
        pltpu.sync_copy(rows0_v, acc_sh.at[pl.ds(base + b * CHUNK, CHUNK)])
    if rem:
        pltpu.sync_copy(rows0_v.at[pl.ds(0, rem)],
                        acc_sh.at[pl.ds(base + nz * CHUNK, rem)])
    plsc.subcore_barrier()

    def blk_body(bb, _):
        pltpu.sync_copy(src_hbm.at[wid].at[pl.ds(bb * IB, IB)], src_v)
        pltpu.sync_copy(dst_hbm.at[wid].at[pl.ds(bb * IB, IB)], dst_v)
        hg = [None, None, None]
        hs = [None, None, None]
        hg[0] = pltpu.async_copy(u_hbm.at[src_v.at[0]], rows[0], sgs[0])
        hg[1] = pltpu.async_copy(u_hbm.at[src_v.at[1]], rows[1], sgs[1])
        for c in range(IB):
            p = c % 3
            hg[p].wait()
            hs[p] = pltpu.async_copy(
                rows[p], acc_sh.at[dst_v.at[c]], sss[p], add=True)
            if c + 2 < IB:
                q = (c + 2) % 3
                if hs[q] is not None:
                    hs[q].wait()
                hg[q] = pltpu.async_copy(
                    u_hbm.at[src_v.at[c + 2]], rows[q], sgs[q])
        hs[(IB - 2) % 3].wait()
        hs[(IB - 1) % 3].wait()
        return 0

    lax.fori_loop(0, CPW // IB, blk_body, 0)
    plsc.subcore_barrier()

    @pl.when(cid == 0)
    def _():
        pltpu.sync_copy(acc_sh.at[pl.ds(base, ROWS_PW)],
                        accA_hbm.at[pl.ds(base, ROWS_PW)])

    @pl.when(cid == 1)
    def _():
        pltpu.sync_copy(acc_sh.at[pl.ds(base, ROWS_PW)],
                        accB_hbm.at[pl.ds(base, ROWS_PW)])


def _row_spec(w):
    return pl.BlockSpec((BR, w), lambda i: (i, 0))


def _const_spec(hgt, w):
    return pl.BlockSpec((hgt, w), lambda i: (0, 0))


def _degmerge_body(a_ref, b_ref, o_ref):
    o_ref[...] = 1.0 / (a_ref[...] + b_ref[...] + 1.0)


def _degmerge(degA, degB):
    return pl.pallas_call(
        _degmerge_body,
        grid=(GRID,),
        in_specs=[_row_spec(16), _row_spec(16)],
        out_specs=_row_spec(16),
        out_shape=jax.ShapeDtypeStruct((NPAD, 16), f32),
    )(degA, degB)


def _mm_scale_body(h_ref, w_ref, b_ref, dg_ref, o_ref):
    acc = jnp.dot(h_ref[...], w_ref[...], preferred_element_type=f32)
    acc = acc + b_ref[...]
    o_ref[...] = jnp.sqrt(dg_ref[...][:, :1]) * acc


def _mm_scale(h, W, b, dg):
    return pl.pallas_call(
        _mm_scale_body,
        grid=(GRID,),
        in_specs=[_row_spec(H), _const_spec(H, H), _const_spec(1, H), _row_spec(16)],
        out_specs=_row_spec(H),
        out_shape=jax.ShapeDtypeStruct((NPAD, H), f32),
    )(h, W, b.reshape(1, H), dg)


def _merge_body(a_ref, b_ref, u_ref, dg_ref, o_ref):
    o_ref[...] = dg_ref[...][:, :1] * (a_ref[...] + b_ref[...] + u_ref[...])


def _merge(accA, accB, u, dg):
    return pl.pallas_call(
        _merge_body,
        grid=(GRID,),
        in_specs=[_row_spec(H), _row_spec(H), _row_spec(H), _row_spec(16)],
        out_specs=_row_spec(H),
        out_shape=jax.ShapeDtypeStruct((NPAD, H), f32),
    )(accA, accB, u, dg)


def _combine_body(u0, u1, u2, u3, u4, s_ref, dg_ref, o_ref):
    s = s_ref[...]
    agg = (u0[...] * s[0:1, :] + u1[...] * s[1:2, :] + u2[...] * s[2:3, :]
           + u3[...] * s[3:4, :] + u4[...] * s[4:5, :])
    o_ref[...] = jnp.maximum(lax.rsqrt(dg_ref[...][:, :1]) * agg, 0.0)


def _combine(us, satt, dg):
    return pl.pallas_call(
        _combine_body,
        grid=(GRID,),
        in_specs=[_row_spec(H)] * 5 + [_const_spec(8, H), _row_spec(16)],
        out_specs=_row_spec(H),
        out_shape=jax.ShapeDtypeStruct((NPAD, H), f32),
    )(*us, satt, dg)


def _out_body(h_ref, w_ref, b_ref, o_ref):
    logits = jnp.dot(h_ref[...], w_ref[...], preferred_element_type=f32)
    logits = logits + b_ref[...]
    m = jnp.max(logits, axis=1, keepdims=True)
    e = jnp.exp(logits - m)
    s = jnp.sum(e, axis=1, keepdims=True)
    o_ref[...] = logits - m - jnp.log(s)


def _out(h, Wp, bp):
    return pl.pallas_call(
        _out_body,
        grid=(GRID,),
        in_specs=[_row_spec(H), _const_spec(H, H), _const_spec(1, H)],
        out_specs=_row_spec(H),
        out_shape=jax.ShapeDtypeStruct((NPAD, H), f32),
    )(h, Wp, bp)


def kernel(x, edge_index, W0, b0, W1, b1, Wout, bout, att, alphas):
    row0 = edge_index[0]
    col0 = edge_index[1]
    dsts = jnp.concatenate([row0, col0])
    srcs = jnp.concatenate([col0, row0])
    pad_e = EPAD - E2
    dst3 = jnp.concatenate(
        [dsts, jnp.full((pad_e,), TRASH, i32)]).reshape(NW, CPW, CHUNK)
    src3 = jnp.concatenate(
        [srcs, jnp.zeros((pad_e,), i32)]).reshape(NW, CPW, CHUNK)

    xp = jnp.zeros((NPAD, D), f32).at[:N].set(x)
    Wp = jnp.zeros((H, 128), f32).at[:, :OUT].set(Wout)
    bp = jnp.full((128,), -1e30, f32).at[:OUT].set(bout).reshape(1, 128)

    al = [ALPHA * jnp.tanh(alphas[j]) for j in range(K + 1)]
    satts = []
    for i in range(NLAYER):
        s = _satt(att[i], al)
        satts.append(jnp.zeros((8, H), f32).at[:K + 1].set(s))

    degA, degB = _deg_k(dst3)
    dg = _degmerge(degA, degB)

    h = xp
    Ws = [(W0, b0), (W1, b1)]
    for i in range(NLAYER):
        Wi, bi = Ws[i]
        u = _mm_scale(h, Wi, bi, dg)
        us = [u]
        for _ in range(K):
            accA, accB = _prop_k(u, src3, dst3)
            u = _merge(accA, accB, u, dg)
            us.append(u)
        h = _combine(us, satts[i], dg)

    res = _out(h, Wp, bp)
    return res[:N, :OUT]


# speedup vs baseline: 2.1703x; 2.1703x over previous
import functools

import jax
import jax.numpy as jnp
from jax import lax
from jax.experimental import pallas as pl
from jax.experimental.pallas import tpu as pltpu
from jax.experimental.pallas import tpu_sc as plsc

N = 10000
D = 128
H = 128
K = 4
NLAYER = 2
OUT = 10
ALPHA = 1.0
PA = 1.0
PB = 1.0

NPAD = 10240
ACC_ROWS = 10016
E2 = 2 * 320000
NW = 32
CHUNK = 128
IB = 4
CPW = -(-E2 // (NW * CHUNK * IB)) * IB
EPAD = NW * CPW * CHUNK
TRASH = ACC_ROWS - 1
W_STRIDE = 632
LAST_ROWS = ACC_ROWS - 15 * W_STRIDE

BR = 1024
GRID = NPAD // BR

f32 = jnp.float32
i32 = jnp.int32


def _shift1(v):
    return jnp.concatenate([jnp.zeros((1,), v.dtype), v[:-1]])


def _expansion(prev, L, al, a, b, l=-1.0, r=1.0):
    if L == 0:
        return prev[0]
    if L == 1:
        coef1 = ((a - b) / 2.0 - (a + b + 2.0) / 2.0 * (l + r) / (r - l)) * al[0]
        coef2 = ((a + b + 2.0) / (r - l)) * al[0]
        t1 = prev[-1]
        return coef1 * t1 + coef2 * _shift1(t1)
    coef_l = 2.0 * L * (L + a + b) * (2.0 * L - 2.0 + a + b)
    coef_lm1_1 = (2.0 * L + a + b - 1.0) * (2.0 * L + a + b) * (2.0 * L + a + b - 2.0)
    coef_lm1_2 = (2.0 * L + a + b - 1.0) * (a ** 2 - b ** 2)
    coef_lm2 = 2.0 * (L - 1.0 + a) * (L - 1.0 + b) * (2.0 * L + a + b)
    tmp1 = al[L - 1] * (coef_lm1_1 / coef_l)
    tmp2 = al[L - 1] * (coef_lm1_2 / coef_l)
    tmp3 = al[L - 1] * al[L - 2] * (coef_lm2 / coef_l)
    tmp1_2 = tmp1 * (2.0 / (r - l))
    tmp2_2 = tmp1 * ((r + l) / (r - l)) + tmp2
    t1 = prev[-1]
    return tmp1_2 * _shift1(t1) - tmp2_2 * t1 - tmp3 * prev[-2]


def _satt(att_i, al):
    tmp0 = jnp.zeros((K + 1,), f32).at[0].set(1.0)
    xs = [tmp0]
    s = att_i[:, 0][:, None] * tmp0[None, :]
    for j in range(1, K + 1):
        tx = _expansion(xs, j, al, PA, PB)
        s = s + att_i[:, j][:, None] * tx[None, :]
        xs.append(tx)
    return s.T


_MESH = plsc.VectorSubcoreMesh(core_axis_name="c", subcore_axis_name="s")


@functools.partial(
    pl.kernel,
    mesh=_MESH,
    out_type=[jax.ShapeDtypeStruct((NPAD, 16), f32),
              jax.ShapeDtypeStruct((NPAD, 16), f32)],
    scratch_types=[
        pltpu.VMEM_SHARED((ACC_ROWS, 16), f32),
        pltpu.VMEM((IB, CHUNK), i32),
        pltpu.VMEM((CHUNK, 16), f32),
        pltpu.VMEM((CHUNK, 16), f32),
    ],
)
def _deg_k(dst_hbm, degA_hbm, degB_hbm, acc_sh, dst_v, ones_v, zeros_v):
    cid = lax.axis_index("c")
    sid = lax.axis_index("s")
    wid = sid * 2 + cid
    base = sid * W_STRIDE

    def fill(rr, _):
        ones_v[rr] = jnp.full((16,), 1.0, f32)
        zeros_v[rr] = jnp.zeros((16,), f32)
        return 0

    lax.fori_loop(0, CHUNK, fill, 0)
    for b in range(4):
        pltpu.sync_copy(zeros_v, acc_sh.at[pl.ds(base + b * 128, 128)])

    @pl.when(sid < 15)
    def _():
        pltpu.sync_copy(zeros_v.at[pl.ds(0, W_STRIDE - 512)],
                        acc_sh.at[pl.ds(base + 512, W_STRIDE - 512)])

    @pl.when(sid == 15)
    def _():
        pltpu.sync_copy(zeros_v.at[pl.ds(0, LAST_ROWS - 512)],
                        acc_sh.at[pl.ds(base + 512, LAST_ROWS - 512)])

    plsc.subcore_barrier()

    def blk_body(bb, _):
        pltpu.sync_copy(dst_hbm.at[wid].at[pl.ds(bb * IB, IB)], dst_v)
        for c in range(IB):
            pltpu.sync_copy(ones_v, acc_sh.at[dst_v.at[c]], add=True)
        return 0

    lax.fori_loop(0, CPW // IB, blk_body, 0)
    plsc.subcore_barrier()

    for last in (False, True):
        @pl.when((sid == 15) == last)
        def _():
            n = LAST_ROWS if last else W_STRIDE

            @pl.when(cid == 0)
            def _():
                pltpu.sync_copy(acc_sh.at[pl.ds(base, n)],
                                degA_hbm.at[pl.ds(base, n)])

            @pl.when(cid == 1)
            def _():
                pltpu.sync_copy(acc_sh.at[pl.ds(base, n)],
                                degB_hbm.at[pl.ds(base, n)])


@functools.partial(
    pl.kernel,
    mesh=_MESH,
    out_type=[jax.ShapeDtypeStruct((NPAD, H), f32),
              jax.ShapeDtypeStruct((NPAD, H), f32)],
    scratch_types=[
        pltpu.VMEM_SHARED((ACC_ROWS, H), f32),
        pltpu.VMEM((IB, CHUNK), i32),
        pltpu.VMEM((IB, CHUNK), i32),
        pltpu.VMEM((CHUNK, H), f32),
        pltpu.VMEM((CHUNK, H), f32),
        pltpu.VMEM((CHUNK, H), f32),
        pltpu.SemaphoreType.DMA,
        pltpu.SemaphoreType.DMA,
        pltpu.SemaphoreType.DMA,
        pltpu.SemaphoreType.DMA,
        pltpu.SemaphoreType.DMA,
        pltpu.SemaphoreType.DMA,
    ],
)
def _prop_k(u_hbm, src_hbm, dst_hbm, accA_hbm, accB_hbm,
            acc_sh, src_v, dst_v, rows0_v, rows1_v, rows2_v,
            sg0, sg1, sg2, ss0, ss1, ss2):
    cid = lax.axis_index("c")
    sid = lax.axis_index("s")
    wid = sid * 2 + cid
    base = sid * W_STRIDE
    zeros16 = jnp.zeros((16,), f32)
    rows = [rows0_v, rows1_v, rows2_v]
    sgs = [sg0, sg1, sg2]
    sss = [ss0, ss1, ss2]

    def zrow(rr, _):
        for k in range(H // 16):
            rows0_v[rr, k * 16:(k + 1) * 16] = zeros16
        return 0

    lax.fori_loop(0, CHUNK, zrow, 0)
    for b in range(4):
        pltpu.sync_copy(rows0_v, acc_sh.at[pl.ds(base + b * 128, 128)])

    @pl.when(sid < 15)
    def _():
        pltpu.sync_copy(rows0_v.at[pl.ds(0, W_STRIDE - 512)],
                        acc_sh.at[pl.ds(base + 512, W_STRIDE - 512)])

    @pl.when(sid == 15)
    def _():
        pltpu.sync_copy(rows0_v.at[pl.ds(0, LAST_ROWS - 512)],
                        acc_sh.at[pl.ds(base + 512, LAST_ROWS - 512)])

    plsc.subcore_barrier()

    def blk_body(bb, _):
        pltpu.sync_copy(src_hbm.at[wid].at[pl.ds(bb * IB, IB)], src_v)
        pltpu.sync_copy(dst_hbm.at[wid].at[pl.ds(bb * IB, IB)], dst_v)
        hg = [None, None, None]
        hs = [None, None, None]
        pending = [False, False, False]
        hg[0] = pltpu.async_copy(u_hbm.at[src_v.at[0]], rows[0], sgs[0])
        if IB > 1:
            hg[1] = pltpu.async_copy(u_hbm.at[src_v.at[1]], rows[1], sgs[1])
        for c in range(IB):
            p = c % 3
            hg[p].wait()
            hs[p] = pltpu.async_copy(
                rows[p], acc_sh.at[dst_v.at[c]], sss[p], add=True)
            pending[p] = True
            if c + 2 < IB:
                q = (c + 2) % 3
                if pending[q]:
                    hs[q].wait()
                    pending[q] = False
                hg[q] = pltpu.async_copy(
                    u_hbm.at[src_v.at[c + 2]], rows[q], sgs[q])
        for q in range(3):
            if pending[q]:
                hs[q].wait()
        return 0

    lax.fori_loop(0, CPW // IB, blk_body, 0)
    plsc.subcore_barrier()

    for last in (False, True):
        @pl.when((sid == 15) == last)
        def _():
            n = LAST_ROWS if last else W_STRIDE

            @pl.when(cid == 0)
            def _():
                pltpu.sync_copy(acc_sh.at[pl.ds(base, n)],
                                accA_hbm.at[pl.ds(base, n)])

            @pl.when(cid == 1)
            def _():
                pltpu.sync_copy(acc_sh.at[pl.ds(base, n)],
                                accB_hbm.at[pl.ds(base, n)])


def _row_spec(w):
    return pl.BlockSpec((BR, w), lambda i: (i, 0))


def _const_spec(hgt, w):
    return pl.BlockSpec((hgt, w), lambda i: (0, 0))


def _degmerge_body(a_ref, b_ref, o_ref):
    o_ref[...] = 1.0 / (a_ref[...] + b_ref[...] + 1.0)


def _degmerge(degA, degB):
    return pl.pallas_call(
        _degmerge_body,
        grid=(GRID,),
        in_specs=[_row_spec(16), _row_spec(16)],
        out_specs=_row_spec(16),
        out_shape=jax.ShapeDtypeStruct((NPAD, 16), f32),
    )(degA, degB)


def _mm_scale_body(h_ref, w_ref, b_ref, dg_ref, o_ref):
    acc = jnp.dot(h_ref[...], w_ref[...], preferred_element_type=f32)
    acc = acc + b_ref[...]
    o_ref[...] = jnp.sqrt(dg_ref[...][:, :1]) * acc


def _mm_scale(h, W, b, dg):
    return pl.pallas_call(
        _mm_scale_body,
        grid=(GRID,),
        in_specs=[_row_spec(H), _const_spec(H, H), _const_spec(1, H), _row_spec(16)],
        out_specs=_row_spec(H),
        out_shape=jax.ShapeDtypeStruct((NPAD, H), f32),
    )(h, W, b.reshape(1, H), dg)


def _merge_body(a_ref, b_ref, u_ref, dg_ref, o_ref):
    o_ref[...] = dg_ref[...][:, :1] * (a_ref[...] + b_ref[...] + u_ref[...])


def _merge(accA, accB, u, dg):
    return pl.pallas_call(
        _merge_body,
        grid=(GRID,),
        in_specs=[_row_spec(H), _row_spec(H), _row_spec(H), _row_spec(16)],
        out_specs=_row_spec(H),
        out_shape=jax.ShapeDtypeStruct((NPAD, H), f32),
    )(accA, accB, u, dg)


def _combine_body(u0, u1, u2, u3, u4, s_ref, dg_ref, o_ref):
    s = s_ref[...]
    agg = (u0[...] * s[0:1, :] + u1[...] * s[1:2, :] + u2[...] * s[2:3, :]
           + u3[...] * s[3:4, :] + u4[...] * s[4:5, :])
    o_ref[...] = jnp.maximum(lax.rsqrt(dg_ref[...][:, :1]) * agg, 0.0)


def _combine(us, satt, dg):
    return pl.pallas_call(
        _combine_body,
        grid=(GRID,),
        in_specs=[_row_spec(H)] * 5 + [_const_spec(8, H), _row_spec(16)],
        out_specs=_row_spec(H),
        out_shape=jax.ShapeDtypeStruct((NPAD, H), f32),
    )(*us, satt, dg)


def _out_body(h_ref, w_ref, b_ref, o_ref):
    logits = jnp.dot(h_ref[...], w_ref[...], preferred_element_type=f32)
    logits = logits + b_ref[...]
    m = jnp.max(logits, axis=1, keepdims=True)
    e = jnp.exp(logits - m)
    s = jnp.sum(e, axis=1, keepdims=True)
    o_ref[...] = logits - m - jnp.log(s)


def _out(h, Wp, bp):
    return pl.pallas_call(
        _out_body,
        grid=(GRID,),
        in_specs=[_row_spec(H), _const_spec(H, H), _const_spec(1, H)],
        out_specs=_row_spec(H),
        out_shape=jax.ShapeDtypeStruct((NPAD, H), f32),
    )(h, Wp, bp)


def kernel(x, edge_index, W0, b0, W1, b1, Wout, bout, att, alphas):
    row0 = edge_index[0]
    col0 = edge_index[1]
    dsts = jnp.concatenate([row0, col0])
    srcs = jnp.concatenate([col0, row0])
    pad_e = EPAD - E2
    dst3 = jnp.concatenate(
        [dsts, jnp.full((pad_e,), TRASH, i32)]).reshape(NW, CPW, CHUNK)
    src3 = jnp.concatenate(
        [srcs, jnp.zeros((pad_e,), i32)]).reshape(NW, CPW, CHUNK)

    xp = jnp.zeros((NPAD, D), f32).at[:N].set(x)
    Wp = jnp.zeros((H, 128), f32).at[:, :OUT].set(Wout)
    bp = jnp.full((128,), -1e30, f32).at[:OUT].set(bout).reshape(1, 128)

    al = [ALPHA * jnp.tanh(alphas[j]) for j in range(K + 1)]
    satts = []
    for i in range(NLAYER):
        s = _satt(att[i], al)
        satts.append(jnp.zeros((8, H), f32).at[:K + 1].set(s))

    degA, degB = _deg_k(dst3)
    dg = _degmerge(degA, degB)

    h = xp
    Ws = [(W0, b0), (W1, b1)]
    for i in range(NLAYER):
        Wi, bi = Ws[i]
        u = _mm_scale(h, Wi, bi, dg)
        us = [u]
        for _ in range(K):
            accA, accB = _prop_k(u, src3, dst3)
            u = _merge(accA, accB, u, dg)
            us.append(u)
        h = _combine(us, satts[i], dg)

    res = _out(h, Wp, bp)
    return res[:N, :OUT]


# speedup vs baseline: 4.9691x; 2.2896x over previous
import functools

import jax
import jax.numpy as jnp
from jax import lax
from jax.experimental import pallas as pl
from jax.experimental.pallas import tpu as pltpu
from jax.experimental.pallas import tpu_sc as plsc

N = 10000
D = 128
H = 128
K = 4
NLAYER = 2
OUT = 10
ALPHA = 1.0
PA = 1.0
PB = 1.0

NPAD = 10240
ACC_ROWS = 10016
E2 = 2 * 320000
NW = 32
CHUNK = 128
IB = 4
CPW = -(-E2 // (NW * CHUNK * IB)) * IB
EPAD = NW * CPW * CHUNK
TRASH = ACC_ROWS - 1
W_STRIDE = 632
LAST_ROWS = ACC_ROWS - 15 * W_STRIDE

BR = 1024
GRID = NPAD // BR

f32 = jnp.float32
i32 = jnp.int32


def _shift1(v):
    return jnp.concatenate([jnp.zeros((1,), v.dtype), v[:-1]])


def _expansion(prev, L, al, a, b, l=-1.0, r=1.0):
    if L == 0:
        return prev[0]
    if L == 1:
        coef1 = ((a - b) / 2.0 - (a + b + 2.0) / 2.0 * (l + r) / (r - l)) * al[0]
        coef2 = ((a + b + 2.0) / (r - l)) * al[0]
        t1 = prev[-1]
        return coef1 * t1 + coef2 * _shift1(t1)
    coef_l = 2.0 * L * (L + a + b) * (2.0 * L - 2.0 + a + b)
    coef_lm1_1 = (2.0 * L + a + b - 1.0) * (2.0 * L + a + b) * (2.0 * L + a + b - 2.0)
    coef_lm1_2 = (2.0 * L + a + b - 1.0) * (a ** 2 - b ** 2)
    coef_lm2 = 2.0 * (L - 1.0 + a) * (L - 1.0 + b) * (2.0 * L + a + b)
    tmp1 = al[L - 1] * (coef_lm1_1 / coef_l)
    tmp2 = al[L - 1] * (coef_lm1_2 / coef_l)
    tmp3 = al[L - 1] * al[L - 2] * (coef_lm2 / coef_l)
    tmp1_2 = tmp1 * (2.0 / (r - l))
    tmp2_2 = tmp1 * ((r + l) / (r - l)) + tmp2
    t1 = prev[-1]
    return tmp1_2 * _shift1(t1) - tmp2_2 * t1 - tmp3 * prev[-2]


def _satt(att_i, al):
    tmp0 = jnp.zeros((K + 1,), f32).at[0].set(1.0)
    xs = [tmp0]
    s = att_i[:, 0][:, None] * tmp0[None, :]
    for j in range(1, K + 1):
        tx = _expansion(xs, j, al, PA, PB)
        s = s + att_i[:, j][:, None] * tx[None, :]
        xs.append(tx)
    return s.T


_MESH = plsc.VectorSubcoreMesh(core_axis_name="c", subcore_axis_name="s")


@functools.partial(
    pl.kernel,
    mesh=_MESH,
    out_type=[jax.ShapeDtypeStruct((NPAD, 16), f32),
              jax.ShapeDtypeStruct((NPAD, 16), f32)],
    scratch_types=[
        pltpu.VMEM_SHARED((ACC_ROWS, 16), f32),
        pltpu.VMEM((IB, CHUNK), i32),
        pltpu.VMEM((CHUNK, 16), f32),
        pltpu.VMEM((CHUNK, 16), f32),
    ],
)
def _deg_k(dst_hbm, degA_hbm, degB_hbm, acc_sh, dst_v, ones_v, zeros_v):
    cid = lax.axis_index("c")
    sid = lax.axis_index("s")
    wid = sid * 2 + cid
    base = sid * W_STRIDE

    def fill(rr, _):
        ones_v[rr] = jnp.full((16,), 1.0, f32)
        zeros_v[rr] = jnp.zeros((16,), f32)
        return 0

    lax.fori_loop(0, CHUNK, fill, 0)
    for b in range(4):
        pltpu.sync_copy(zeros_v, acc_sh.at[pl.ds(base + b * 128, 128)])

    @pl.when(sid < 15)
    def _():
        pltpu.sync_copy(zeros_v.at[pl.ds(0, W_STRIDE - 512)],
                        acc_sh.at[pl.ds(base + 512, W_STRIDE - 512)])

    @pl.when(sid == 15)
    def _():
        pltpu.sync_copy(zeros_v.at[pl.ds(0, LAST_ROWS - 512)],
                        acc_sh.at[pl.ds(base + 512, LAST_ROWS - 512)])

    plsc.subcore_barrier()

    def blk_body(bb, _):
        pltpu.sync_copy(dst_hbm.at[wid].at[pl.ds(bb * IB, IB)], dst_v)
        for c in range(IB):
            pltpu.sync_copy(ones_v, acc_sh.at[dst_v.at[c]], add=True)
        return 0

    lax.fori_loop(0, CPW // IB, blk_body, 0)
    plsc.subcore_barrier()

    for last in (False, True):
        @pl.when((sid == 15) == last)
        def _():
            n = LAST_ROWS if last else W_STRIDE

            @pl.when(cid == 0)
            def _():
                pltpu.sync_copy(acc_sh.at[pl.ds(base, n)],
                                degA_hbm.at[pl.ds(base, n)])

            @pl.when(cid == 1)
            def _():
                pltpu.sync_copy(acc_sh.at[pl.ds(base, n)],
                                degB_hbm.at[pl.ds(base, n)])


HH = H // 2


@functools.partial(
    pl.kernel,
    mesh=_MESH,
    out_type=[jax.ShapeDtypeStruct((NPAD, HH), f32),
              jax.ShapeDtypeStruct((NPAD, HH), f32),
              jax.ShapeDtypeStruct((NPAD, HH), f32),
              jax.ShapeDtypeStruct((NPAD, HH), f32)],
    scratch_types=[
        pltpu.VMEM_SHARED((ACC_ROWS, HH), f32),
        pltpu.VMEM_SHARED((ACC_ROWS, HH), f32),
        pltpu.VMEM((IB, CHUNK), i32),
        pltpu.VMEM((IB, CHUNK), i32),
        pltpu.VMEM((CHUNK, HH), f32),
        pltpu.VMEM((CHUNK, HH), f32),
        pltpu.VMEM((CHUNK, HH), f32),
        pltpu.SemaphoreType.DMA,
        pltpu.SemaphoreType.DMA,
        pltpu.SemaphoreType.DMA,
        pltpu.SemaphoreType.DMA,
        pltpu.SemaphoreType.DMA,
        pltpu.SemaphoreType.DMA,
    ],
)
def _prop_k(uL_hbm, uR_hbm, src_hbm, dst_hbm,
            accAL_hbm, accAR_hbm, accBL_hbm, accBR_hbm,
            u_sh, acc_sh, src_v, dst_v, rows0_v, rows1_v, rows2_v,
            sg0, sg1, sg2, ss0, ss1, ss2):
    cid = lax.axis_index("c")
    sid = lax.axis_index("s")
    wid = sid * 2 + cid
    base = sid * W_STRIDE
    zeros16 = jnp.zeros((16,), f32)
    rows = [rows0_v, rows1_v, rows2_v]
    sgs = [sg0, sg1, sg2]
    sss = [ss0, ss1, ss2]

    def zrow(rr, _):
        for k in range(HH // 16):
            rows0_v[rr, k * 16:(k + 1) * 16] = zeros16
        return 0

    lax.fori_loop(0, CHUNK, zrow, 0)

    for h, (u_half, outA, outB) in enumerate(
            [(uL_hbm, accAL_hbm, accBL_hbm), (uR_hbm, accAR_hbm, accBR_hbm)]):
        for last in (False, True):
            @pl.when((sid == 15) == last)
            def _():
                n = LAST_ROWS if last else W_STRIDE
                pltpu.sync_copy(u_half.at[pl.ds(base, n)],
                                u_sh.at[pl.ds(base, n)])
                for b in range(4):
                    pltpu.sync_copy(rows0_v,
                                    acc_sh.at[pl.ds(base + b * 128, 128)])
                pltpu.sync_copy(rows0_v.at[pl.ds(0, n - 512)],
                                acc_sh.at[pl.ds(base + 512, n - 512)])

        plsc.subcore_barrier()

        def blk_body(bb, _):
            pltpu.sync_copy(src_hbm.at[wid].at[pl.ds(bb * IB, IB)], src_v)
            pltpu.sync_copy(dst_hbm.at[wid].at[pl.ds(bb * IB, IB)], dst_v)
            hg = [None, None, None]
            hs = [None, None, None]
            pending = [False, False, False]
            hg[0] = pltpu.async_copy(u_sh.at[src_v.at[0]], rows[0], sgs[0])
            if IB > 1:
                hg[1] = pltpu.async_copy(u_sh.at[src_v.at[1]], rows[1], sgs[1])
            for c in range(IB):
                p = c % 3
                hg[p].wait()
                hs[p] = pltpu.async_copy(
                    rows[p], acc_sh.at[dst_v.at[c]], sss[p], add=True)
                pending[p] = True
                if c + 2 < IB:
                    q = (c + 2) % 3
                    if pending[q]:
                        hs[q].wait()
                        pending[q] = False
                    hg[q] = pltpu.async_copy(
                        u_sh.at[src_v.at[c + 2]], rows[q], sgs[q])
            for q in range(3):
                if pending[q]:
                    hs[q].wait()
            return 0

        lax.fori_loop(0, CPW // IB, blk_body, 0)
        plsc.subcore_barrier()

        for last in (False, True):
            @pl.when((sid == 15) == last)
            def _():
                n = LAST_ROWS if last else W_STRIDE

                @pl.when(cid == 0)
                def _():
                    pltpu.sync_copy(acc_sh.at[pl.ds(base, n)],
                                    outA.at[pl.ds(base, n)])

                @pl.when(cid == 1)
                def _():
                    pltpu.sync_copy(acc_sh.at[pl.ds(base, n)],
                                    outB.at[pl.ds(base, n)])

        if h == 0:
            plsc.subcore_barrier()


def _row_spec(w):
    return pl.BlockSpec((BR, w), lambda i: (i, 0))


def _const_spec(hgt, w):
    return pl.BlockSpec((hgt, w), lambda i: (0, 0))


def _degmerge_body(a_ref, b_ref, o_ref):
    o_ref[...] = 1.0 / (a_ref[...] + b_ref[...] + 1.0)


def _degmerge(degA, degB):
    return pl.pallas_call(
        _degmerge_body,
        grid=(GRID,),
        in_specs=[_row_spec(16), _row_spec(16)],
        out_specs=_row_spec(16),
        out_shape=jax.ShapeDtypeStruct((NPAD, 16), f32),
    )(degA, degB)


def _mm_scale_body(h_ref, w_ref, b_ref, dg_ref, o_ref, oL_ref, oR_ref):
    acc = jnp.dot(h_ref[...], w_ref[...], preferred_element_type=f32)
    acc = acc + b_ref[...]
    u = jnp.sqrt(dg_ref[...][:, :1]) * acc
    o_ref[...] = u
    oL_ref[...] = u[:, :HH]
    oR_ref[...] = u[:, HH:]


def _mm_scale(h, W, b, dg):
    return pl.pallas_call(
        _mm_scale_body,
        grid=(GRID,),
        in_specs=[_row_spec(H), _const_spec(H, H), _const_spec(1, H), _row_spec(16)],
        out_specs=[_row_spec(H), _row_spec(HH), _row_spec(HH)],
        out_shape=[jax.ShapeDtypeStruct((NPAD, H), f32),
                   jax.ShapeDtypeStruct((NPAD, HH), f32),
                   jax.ShapeDtypeStruct((NPAD, HH), f32)],
    )(h, W, b.reshape(1, H), dg)


def _merge_body(aL_ref, aR_ref, bL_ref, bR_ref, u_ref, dg_ref,
                o_ref, oL_ref, oR_ref):
    a = jnp.concatenate([aL_ref[...], aR_ref[...]], axis=1)
    b = jnp.concatenate([bL_ref[...], bR_ref[...]], axis=1)
    un = dg_ref[...][:, :1] * (a + b + u_ref[...])
    o_ref[...] = un
    oL_ref[...] = un[:, :HH]
    oR_ref[...] = un[:, HH:]


def _merge(accAL, accAR, accBL, accBR, u, dg):
    return pl.pallas_call(
        _merge_body,
        grid=(GRID,),
        in_specs=[_row_spec(HH), _row_spec(HH), _row_spec(HH), _row_spec(HH),
                  _row_spec(H), _row_spec(16)],
        out_specs=[_row_spec(H), _row_spec(HH), _row_spec(HH)],
        out_shape=[jax.ShapeDtypeStruct((NPAD, H), f32),
                   jax.ShapeDtypeStruct((NPAD, HH), f32),
                   jax.ShapeDtypeStruct((NPAD, HH), f32)],
    )(accAL, accAR, accBL, accBR, u, dg)


def _combine_body(u0, u1, u2, u3, u4, s_ref, dg_ref, o_ref):
    s = s_ref[...]
    agg = (u0[...] * s[0:1, :] + u1[...] * s[1:2, :] + u2[...] * s[2:3, :]
           + u3[...] * s[3:4, :] + u4[...] * s[4:5, :])
    o_ref[...] = jnp.maximum(lax.rsqrt(dg_ref[...][:, :1]) * agg, 0.0)


def _combine(us, satt, dg):
    return pl.pallas_call(
        _combine_body,
        grid=(GRID,),
        in_specs=[_row_spec(H)] * 5 + [_const_spec(8, H), _row_spec(16)],
        out_specs=_row_spec(H),
        out_shape=jax.ShapeDtypeStruct((NPAD, H), f32),
    )(*us, satt, dg)


def _out_body(h_ref, w_ref, b_ref, o_ref):
    logits = jnp.dot(h_ref[...], w_ref[...], preferred_element_type=f32)
    logits = logits + b_ref[...]
    m = jnp.max(logits, axis=1, keepdims=True)
    e = jnp.exp(logits - m)
    s = jnp.sum(e, axis=1, keepdims=True)
    o_ref[...] = logits - m - jnp.log(s)


def _out(h, Wp, bp):
    return pl.pallas_call(
        _out_body,
        grid=(GRID,),
        in_specs=[_row_spec(H), _const_spec(H, H), _const_spec(1, H)],
        out_specs=_row_spec(H),
        out_shape=jax.ShapeDtypeStruct((NPAD, H), f32),
    )(h, Wp, bp)


def kernel(x, edge_index, W0, b0, W1, b1, Wout, bout, att, alphas):
    row0 = edge_index[0]
    col0 = edge_index[1]
    dsts = jnp.concatenate([row0, col0])
    srcs = jnp.concatenate([col0, row0])
    pad_e = EPAD - E2
    dst3 = jnp.concatenate(
        [dsts, jnp.full((pad_e,), TRASH, i32)]).reshape(NW, CPW, CHUNK)
    src3 = jnp.concatenate(
        [srcs, jnp.zeros((pad_e,), i32)]).reshape(NW, CPW, CHUNK)

    xp = jnp.zeros((NPAD, D), f32).at[:N].set(x)
    Wp = jnp.zeros((H, 128), f32).at[:, :OUT].set(Wout)
    bp = jnp.full((128,), -1e30, f32).at[:OUT].set(bout).reshape(1, 128)

    al = [ALPHA * jnp.tanh(alphas[j]) for j in range(K + 1)]
    satts = []
    for i in range(NLAYER):
        s = _satt(att[i], al)
        satts.append(jnp.zeros((8, H), f32).at[:K + 1].set(s))

    degA, degB = _deg_k(dst3)
    dg = _degmerge(degA, degB)

    h = xp
    Ws = [(W0, b0), (W1, b1)]
    for i in range(NLAYER):
        Wi, bi = Ws[i]
        u, uL, uR = _mm_scale(h, Wi, bi, dg)
        us = [u]
        for _ in range(K):
            aAL, aAR, aBL, aBR = _prop_k(uL, uR, src3, dst3)
            u, uL, uR = _merge(aAL, aAR, aBL, aBR, u, dg)
            us.append(u)
        h = _combine(us, satts[i], dg)

    res = _out(h, Wp, bp)
    return res[:N, :OUT]


# speedup vs baseline: 5.9325x; 1.1939x over previous
import functools

import jax
import jax.numpy as jnp
from jax import lax
from jax.experimental import pallas as pl
from jax.experimental.pallas import tpu as pltpu
from jax.experimental.pallas import tpu_sc as plsc

N = 10000
D = 128
H = 128
K = 4
NLAYER = 2
OUT = 10
ALPHA = 1.0
PA = 1.0
PB = 1.0

NPAD = 10240
ACC_ROWS = 10016
E2 = 2 * 320000
NW = 32
CHUNK = 128
IB = 32
CPW = -(-E2 // (NW * CHUNK * IB)) * IB
EPAD = NW * CPW * CHUNK
TRASH = ACC_ROWS - 1
W_STRIDE = 632
LAST_ROWS = ACC_ROWS - 15 * W_STRIDE

BR = 1024
GRID = NPAD // BR

f32 = jnp.float32
i32 = jnp.int32


def _shift1(v):
    return jnp.concatenate([jnp.zeros((1,), v.dtype), v[:-1]])


def _expansion(prev, L, al, a, b, l=-1.0, r=1.0):
    if L == 0:
        return prev[0]
    if L == 1:
        coef1 = ((a - b) / 2.0 - (a + b + 2.0) / 2.0 * (l + r) / (r - l)) * al[0]
        coef2 = ((a + b + 2.0) / (r - l)) * al[0]
        t1 = prev[-1]
        return coef1 * t1 + coef2 * _shift1(t1)
    coef_l = 2.0 * L * (L + a + b) * (2.0 * L - 2.0 + a + b)
    coef_lm1_1 = (2.0 * L + a + b - 1.0) * (2.0 * L + a + b) * (2.0 * L + a + b - 2.0)
    coef_lm1_2 = (2.0 * L + a + b - 1.0) * (a ** 2 - b ** 2)
    coef_lm2 = 2.0 * (L - 1.0 + a) * (L - 1.0 + b) * (2.0 * L + a + b)
    tmp1 = al[L - 1] * (coef_lm1_1 / coef_l)
    tmp2 = al[L - 1] * (coef_lm1_2 / coef_l)
    tmp3 = al[L - 1] * al[L - 2] * (coef_lm2 / coef_l)
    tmp1_2 = tmp1 * (2.0 / (r - l))
    tmp2_2 = tmp1 * ((r + l) / (r - l)) + tmp2
    t1 = prev[-1]
    return tmp1_2 * _shift1(t1) - tmp2_2 * t1 - tmp3 * prev[-2]


def _satt(att_i, al):
    tmp0 = jnp.zeros((K + 1,), f32).at[0].set(1.0)
    xs = [tmp0]
    s = att_i[:, 0][:, None] * tmp0[None, :]
    for j in range(1, K + 1):
        tx = _expansion(xs, j, al, PA, PB)
        s = s + att_i[:, j][:, None] * tx[None, :]
        xs.append(tx)
    return s.T


_MESH = plsc.VectorSubcoreMesh(core_axis_name="c", subcore_axis_name="s")


@functools.partial(
    pl.kernel,
    mesh=_MESH,
    out_type=[jax.ShapeDtypeStruct((NPAD, 16), f32),
              jax.ShapeDtypeStruct((NPAD, 16), f32)],
    scratch_types=[
        pltpu.VMEM_SHARED((ACC_ROWS, 16), f32),
        pltpu.VMEM((IB, CHUNK), i32),
        pltpu.VMEM((CHUNK, 16), f32),
        pltpu.VMEM((CHUNK, 16), f32),
    ],
)
def _deg_k(dst_hbm, degA_hbm, degB_hbm, acc_sh, dst_v, ones_v, zeros_v):
    cid = lax.axis_index("c")
    sid = lax.axis_index("s")
    wid = sid * 2 + cid
    base = sid * W_STRIDE

    def fill(rr, _):
        ones_v[rr] = jnp.full((16,), 1.0, f32)
        zeros_v[rr] = jnp.zeros((16,), f32)
        return 0

    lax.fori_loop(0, CHUNK, fill, 0)
    for b in range(4):
        pltpu.sync_copy(zeros_v, acc_sh.at[pl.ds(base + b * 128, 128)])

    @pl.when(sid < 15)
    def _():
        pltpu.sync_copy(zeros_v.at[pl.ds(0, W_STRIDE - 512)],
                        acc_sh.at[pl.ds(base + 512, W_STRIDE - 512)])

    @pl.when(sid == 15)
    def _():
        pltpu.sync_copy(zeros_v.at[pl.ds(0, LAST_ROWS - 512)],
                        acc_sh.at[pl.ds(base + 512, LAST_ROWS - 512)])

    plsc.subcore_barrier()

    def blk_body(bb, _):
        pltpu.sync_copy(dst_hbm.at[wid].at[pl.ds(bb * IB, IB)], dst_v)
        for c in range(IB):
            pltpu.sync_copy(ones_v, acc_sh.at[dst_v.at[c]], add=True)
        return 0

    lax.fori_loop(0, CPW // IB, blk_body, 0)
    plsc.subcore_barrier()

    for last in (False, True):
        @pl.when((sid == 15) == last)
        def _():
            n = LAST_ROWS if last else W_STRIDE

            @pl.when(cid == 0)
            def _():
                pltpu.sync_copy(acc_sh.at[pl.ds(base, n)],
                                degA_hbm.at[pl.ds(base, n)])

            @pl.when(cid == 1)
            def _():
                pltpu.sync_copy(acc_sh.at[pl.ds(base, n)],
                                degB_hbm.at[pl.ds(base, n)])


HH = H // 2


@functools.partial(
    pl.kernel,
    mesh=_MESH,
    out_type=[jax.ShapeDtypeStruct((NPAD, HH), f32),
              jax.ShapeDtypeStruct((NPAD, HH), f32),
              jax.ShapeDtypeStruct((NPAD, HH), f32),
              jax.ShapeDtypeStruct((NPAD, HH), f32)],
    scratch_types=[
        pltpu.VMEM_SHARED((ACC_ROWS, HH), f32),
        pltpu.VMEM_SHARED((ACC_ROWS, HH), f32),
        pltpu.VMEM((IB, CHUNK), i32),
        pltpu.VMEM((IB, CHUNK), i32),
        pltpu.VMEM((CHUNK, HH), f32),
        pltpu.VMEM((CHUNK, HH), f32),
        pltpu.SemaphoreType.DMA,
        pltpu.SemaphoreType.DMA,
        pltpu.SemaphoreType.DMA,
        pltpu.SemaphoreType.DMA,
    ],
)
def _prop_k(uL_hbm, uR_hbm, src_hbm, dst_hbm,
            accAL_hbm, accAR_hbm, accBL_hbm, accBR_hbm,
            u_sh, acc_sh, src_v, dst_v, rows0_v, rows1_v,
            sg0, sg1, ss0, ss1):
    cid = lax.axis_index("c")
    sid = lax.axis_index("s")
    wid = sid * 2 + cid
    base = sid * W_STRIDE
    zeros16 = jnp.zeros((16,), f32)
    rows = [rows0_v, rows1_v]
    sgs = [sg0, sg1]
    sss = [ss0, ss1]

    def zrow(rr, _):
        for k in range(HH // 16):
            rows0_v[rr, k * 16:(k + 1) * 16] = zeros16
        return 0

    lax.fori_loop(0, CHUNK, zrow, 0)

    for h, (u_half, outA, outB) in enumerate(
            [(uL_hbm, accAL_hbm, accBL_hbm), (uR_hbm, accAR_hbm, accBR_hbm)]):
        for last in (False, True):
            @pl.when((sid == 15) == last)
            def _():
                n = LAST_ROWS if last else W_STRIDE
                pltpu.sync_copy(u_half.at[pl.ds(base, n)],
                                u_sh.at[pl.ds(base, n)])
                for b in range(4):
                    pltpu.sync_copy(rows0_v,
                                    acc_sh.at[pl.ds(base + b * 128, 128)])
                pltpu.sync_copy(rows0_v.at[pl.ds(0, n - 512)],
                                acc_sh.at[pl.ds(base + 512, n - 512)])

        plsc.subcore_barrier()

        def blk_body(bb, _):
            pltpu.sync_copy(src_hbm.at[wid].at[pl.ds(bb * IB, IB)], src_v)
            pltpu.sync_copy(dst_hbm.at[wid].at[pl.ds(bb * IB, IB)], dst_v)
            hg = [None, None]
            hs = [None, None]
            pending = [False, False]
            hg[0] = pltpu.async_copy(u_sh.at[src_v.at[0]], rows[0], sgs[0])
            for c in range(IB):
                p = c % 2
                q = (c + 1) % 2
                hg[p].wait()
                hs[p] = pltpu.async_copy(
                    rows[p], acc_sh.at[dst_v.at[c]], sss[p], add=True)
                pending[p] = True
                if c + 1 < IB:
                    if pending[q]:
                        hs[q].wait()
                        pending[q] = False
                    hg[q] = pltpu.async_copy(
                        u_sh.at[src_v.at[c + 1]], rows[q], sgs[q])
            for q in range(2):
                if pending[q]:
                    hs[q].wait()
            return 0

        lax.fori_loop(0, CPW // IB, blk_body, 0)
        plsc.subcore_barrier()

        for last in (False, True):
            @pl.when((sid == 15) == last)
            def _():
                n = LAST_ROWS if last else W_STRIDE

                @pl.when(cid == 0)
                def _():
                    pltpu.sync_copy(acc_sh.at[pl.ds(base, n)],
                                    outA.at[pl.ds(base, n)])

                @pl.when(cid == 1)
                def _():
                    pltpu.sync_copy(acc_sh.at[pl.ds(base, n)],
                                    outB.at[pl.ds(base, n)])

        if h == 0:
            plsc.subcore_barrier()


def _row_spec(w):
    return pl.BlockSpec((BR, w), lambda i: (i, 0))


def _const_spec(hgt, w):
    return pl.BlockSpec((hgt, w), lambda i: (0, 0))


def _degmerge_body(a_ref, b_ref, o_ref):
    o_ref[...] = 1.0 / (a_ref[...] + b_ref[...] + 1.0)


def _degmerge(degA, degB):
    return pl.pallas_call(
        _degmerge_body,
        grid=(GRID,),
        in_specs=[_row_spec(16), _row_spec(16)],
        out_specs=_row_spec(16),
        out_shape=jax.ShapeDtypeStruct((NPAD, 16), f32),
    )(degA, degB)


def _mm_scale_body(h_ref, w_ref, b_ref, dg_ref, o_ref, oL_ref, oR_ref):
    acc = jnp.dot(h_ref[...], w_ref[...], preferred_element_type=f32)
    acc = acc + b_ref[...]
    u = jnp.sqrt(dg_ref[...][:, :1]) * acc
    o_ref[...] = u
    oL_ref[...] = u[:, :HH]
    oR_ref[...] = u[:, HH:]


def _mm_scale(h, W, b, dg):
    return pl.pallas_call(
        _mm_scale_body,
        grid=(GRID,),
        in_specs=[_row_spec(H), _const_spec(H, H), _const_spec(1, H), _row_spec(16)],
        out_specs=[_row_spec(H), _row_spec(HH), _row_spec(HH)],
        out_shape=[jax.ShapeDtypeStruct((NPAD, H), f32),
                   jax.ShapeDtypeStruct((NPAD, HH), f32),
                   jax.ShapeDtypeStruct((NPAD, HH), f32)],
    )(h, W, b.reshape(1, H), dg)


def _merge_body(aL_ref, aR_ref, bL_ref, bR_ref, u_ref, dg_ref,
                o_ref, oL_ref, oR_ref):
    a = jnp.concatenate([aL_ref[...], aR_ref[...]], axis=1)
    b = jnp.concatenate([bL_ref[...], bR_ref[...]], axis=1)
    un = dg_ref[...][:, :1] * (a + b + u_ref[...])
    o_ref[...] = un
    oL_ref[...] = un[:, :HH]
    oR_ref[...] = un[:, HH:]


def _merge(accAL, accAR, accBL, accBR, u, dg):
    return pl.pallas_call(
        _merge_body,
        grid=(GRID,),
        in_specs=[_row_spec(HH), _row_spec(HH), _row_spec(HH), _row_spec(HH),
                  _row_spec(H), _row_spec(16)],
        out_specs=[_row_spec(H), _row_spec(HH), _row_spec(HH)],
        out_shape=[jax.ShapeDtypeStruct((NPAD, H), f32),
                   jax.ShapeDtypeStruct((NPAD, HH), f32),
                   jax.ShapeDtypeStruct((NPAD, HH), f32)],
    )(accAL, accAR, accBL, accBR, u, dg)


def _combine_body(u0, u1, u2, u3, u4, s_ref, dg_ref, o_ref):
    s = s_ref[...]
    agg = (u0[...] * s[0:1, :] + u1[...] * s[1:2, :] + u2[...] * s[2:3, :]
           + u3[...] * s[3:4, :] + u4[...] * s[4:5, :])
    o_ref[...] = jnp.maximum(lax.rsqrt(dg_ref[...][:, :1]) * agg, 0.0)


def _combine(us, satt, dg):
    return pl.pallas_call(
        _combine_body,
        grid=(GRID,),
        in_specs=[_row_spec(H)] * 5 + [_const_spec(8, H), _row_spec(16)],
        out_specs=_row_spec(H),
        out_shape=jax.ShapeDtypeStruct((NPAD, H), f32),
    )(*us, satt, dg)


def _out_body(h_ref, w_ref, b_ref, o_ref):
    logits = jnp.dot(h_ref[...], w_ref[...], preferred_element_type=f32)
    logits = logits + b_ref[...]
    m = jnp.max(logits, axis=1, keepdims=True)
    e = jnp.exp(logits - m)
    s = jnp.sum(e, axis=1, keepdims=True)
    o_ref[...] = logits - m - jnp.log(s)


def _out(h, Wp, bp):
    return pl.pallas_call(
        _out_body,
        grid=(GRID,),
        in_specs=[_row_spec(H), _const_spec(H, H), _const_spec(1, H)],
        out_specs=_row_spec(H),
        out_shape=jax.ShapeDtypeStruct((NPAD, H), f32),
    )(h, Wp, bp)


def kernel(x, edge_index, W0, b0, W1, b1, Wout, bout, att, alphas):
    row0 = edge_index[0]
    col0 = edge_index[1]
    dsts = jnp.concatenate([row0, col0])
    srcs = jnp.concatenate([col0, row0])
    pad_e = EPAD - E2
    dst3 = jnp.concatenate(
        [dsts, jnp.full((pad_e,), TRASH, i32)]).reshape(NW, CPW, CHUNK)
    src3 = jnp.concatenate(
        [srcs, jnp.zeros((pad_e,), i32)]).reshape(NW, CPW, CHUNK)

    xp = jnp.zeros((NPAD, D), f32).at[:N].set(x)
    Wp = jnp.zeros((H, 128), f32).at[:, :OUT].set(Wout)
    bp = jnp.full((128,), -1e30, f32).at[:OUT].set(bout).reshape(1, 128)

    al = [ALPHA * jnp.tanh(alphas[j]) for j in range(K + 1)]
    satts = []
    for i in range(NLAYER):
        s = _satt(att[i], al)
        satts.append(jnp.zeros((8, H), f32).at[:K + 1].set(s))

    degA, degB = _deg_k(dst3)
    dg = _degmerge(degA, degB)

    h = xp
    Ws = [(W0, b0), (W1, b1)]
    for i in range(NLAYER):
        Wi, bi = Ws[i]
        u, uL, uR = _mm_scale(h, Wi, bi, dg)
        us = [u]
        for _ in range(K):
            aAL, aAR, aBL, aBR = _prop_k(uL, uR, src3, dst3)
            u, uL, uR = _merge(aAL, aAR, aBL, aBR, u, dg)
            us.append(u)
        h = _combine(us, satts[i], dg)

    res = _out(h, Wp, bp)
    return res[:N, :OUT]


# speedup vs baseline: 5.9327x; 1.0000x over previous
import functools

import jax
import jax.numpy as jnp
from jax import lax
from jax.experimental import pallas as pl
from jax.experimental.pallas import tpu as pltpu
from jax.experimental.pallas import tpu_sc as plsc

N = 10000
D = 128
H = 128
K = 4
NLAYER = 2
OUT = 10
ALPHA = 1.0
PA = 1.0
PB = 1.0

NPAD = 10240
ACC_ROWS = 10016
E2 = 2 * 320000
NW = 32
CHUNK = 128
IB = 32
CPW = -(-E2 // (NW * CHUNK * IB)) * IB
EPAD = NW * CPW * CHUNK
TRASH = ACC_ROWS - 1
W_STRIDE = 632
LAST_ROWS = ACC_ROWS - 15 * W_STRIDE

BR = 1024
GRID = NPAD // BR

f32 = jnp.float32
i32 = jnp.int32


def _shift1(v):
    return jnp.concatenate([jnp.zeros((1,), v.dtype), v[:-1]])


def _expansion(prev, L, al, a, b, l=-1.0, r=1.0):
    if L == 0:
        return prev[0]
    if L == 1:
        coef1 = ((a - b) / 2.0 - (a + b + 2.0) / 2.0 * (l + r) / (r - l)) * al[0]
        coef2 = ((a + b + 2.0) / (r - l)) * al[0]
        t1 = prev[-1]
        return coef1 * t1 + coef2 * _shift1(t1)
    coef_l = 2.0 * L * (L + a + b) * (2.0 * L - 2.0 + a + b)
    coef_lm1_1 = (2.0 * L + a + b - 1.0) * (2.0 * L + a + b) * (2.0 * L + a + b - 2.0)
    coef_lm1_2 = (2.0 * L + a + b - 1.0) * (a ** 2 - b ** 2)
    coef_lm2 = 2.0 * (L - 1.0 + a) * (L - 1.0 + b) * (2.0 * L + a + b)
    tmp1 = al[L - 1] * (coef_lm1_1 / coef_l)
    tmp2 = al[L - 1] * (coef_lm1_2 / coef_l)
    tmp3 = al[L - 1] * al[L - 2] * (coef_lm2 / coef_l)
    tmp1_2 = tmp1 * (2.0 / (r - l))
    tmp2_2 = tmp1 * ((r + l) / (r - l)) + tmp2
    t1 = prev[-1]
    return tmp1_2 * _shift1(t1) - tmp2_2 * t1 - tmp3 * prev[-2]


def _satt(att_i, al):
    tmp0 = jnp.zeros((K + 1,), f32).at[0].set(1.0)
    xs = [tmp0]
    s = att_i[:, 0][:, None] * tmp0[None, :]
    for j in range(1, K + 1):
        tx = _expansion(xs, j, al, PA, PB)
        s = s + att_i[:, j][:, None] * tx[None, :]
        xs.append(tx)
    return s.T


_MESH = plsc.VectorSubcoreMesh(core_axis_name="c", subcore_axis_name="s")


@functools.partial(
    pl.kernel,
    mesh=_MESH,
    out_type=[jax.ShapeDtypeStruct((NPAD, 16), f32),
              jax.ShapeDtypeStruct((NPAD, 16), f32)],
    scratch_types=[
        pltpu.VMEM_SHARED((ACC_ROWS, 16), f32),
        pltpu.VMEM((IB, CHUNK), i32),
        pltpu.VMEM((CHUNK, 16), f32),
        pltpu.VMEM((CHUNK, 16), f32),
        pltpu.SemaphoreType.DMA,
    ],
)
def _deg_k(dst_hbm, degA_hbm, degB_hbm, acc_sh, dst_v, ones_v, zeros_v, sd):
    cid = lax.axis_index("c")
    sid = lax.axis_index("s")
    wid = sid * 2 + cid
    base = sid * W_STRIDE

    def fill(rr, _):
        ones_v[rr] = jnp.full((16,), 1.0, f32)
        zeros_v[rr] = jnp.zeros((16,), f32)
        return 0

    lax.fori_loop(0, CHUNK, fill, 0)
    for b in range(4):
        pltpu.sync_copy(zeros_v, acc_sh.at[pl.ds(base + b * 128, 128)])

    @pl.when(sid < 15)
    def _():
        pltpu.sync_copy(zeros_v.at[pl.ds(0, W_STRIDE - 512)],
                        acc_sh.at[pl.ds(base + 512, W_STRIDE - 512)])

    @pl.when(sid == 15)
    def _():
        pltpu.sync_copy(zeros_v.at[pl.ds(0, LAST_ROWS - 512)],
                        acc_sh.at[pl.ds(base + 512, LAST_ROWS - 512)])

    plsc.subcore_barrier()

    def blk_body(bb, _):
        pltpu.sync_copy(dst_hbm.at[wid].at[pl.ds(bb * IB, IB)], dst_v)
        hh = [pltpu.async_copy(ones_v, acc_sh.at[dst_v.at[c]], sd, add=True)
              for c in range(IB)]
        for handle in hh:
            handle.wait()
        return 0

    lax.fori_loop(0, CPW // IB, blk_body, 0)
    plsc.subcore_barrier()

    for last in (False, True):
        @pl.when((sid == 15) == last)
        def _():
            n = LAST_ROWS if last else W_STRIDE

            @pl.when(cid == 0)
            def _():
                pltpu.sync_copy(acc_sh.at[pl.ds(base, n)],
                                degA_hbm.at[pl.ds(base, n)])

            @pl.when(cid == 1)
            def _():
                pltpu.sync_copy(acc_sh.at[pl.ds(base, n)],
                                degB_hbm.at[pl.ds(base, n)])


HH = H // 2


@functools.partial(
    pl.kernel,
    mesh=_MESH,
    out_type=[jax.ShapeDtypeStruct((NPAD, HH), f32),
              jax.ShapeDtypeStruct((NPAD, HH), f32),
              jax.ShapeDtypeStruct((NPAD, HH), f32),
              jax.ShapeDtypeStruct((NPAD, HH), f32)],
    scratch_types=[
        pltpu.VMEM_SHARED((ACC_ROWS, HH), f32),
        pltpu.VMEM_SHARED((ACC_ROWS, HH), f32),
        pltpu.VMEM((IB, CHUNK), i32),
        pltpu.VMEM((IB, CHUNK), i32),
        pltpu.VMEM((CHUNK, HH), f32),
        pltpu.VMEM((CHUNK, HH), f32),
        pltpu.SemaphoreType.DMA,
        pltpu.SemaphoreType.DMA,
        pltpu.SemaphoreType.DMA,
        pltpu.SemaphoreType.DMA,
    ],
)
def _prop_k(uL_hbm, uR_hbm, src_hbm, dst_hbm,
            accAL_hbm, accAR_hbm, accBL_hbm, accBR_hbm,
            u_sh, acc_sh, src_v, dst_v, rows0_v, rows1_v,
            sg0, sg1, ss0, ss1):
    cid = lax.axis_index("c")
    sid = lax.axis_index("s")
    wid = sid * 2 + cid
    base = sid * W_STRIDE
    zeros16 = jnp.zeros((16,), f32)
    rows = [rows0_v, rows1_v]
    sgs = [sg0, sg1]
    sss = [ss0, ss1]

    def zrow(rr, _):
        for k in range(HH // 16):
            rows0_v[rr, k * 16:(k + 1) * 16] = zeros16
        return 0

    lax.fori_loop(0, CHUNK, zrow, 0)

    for h, (u_half, outA, outB) in enumerate(
            [(uL_hbm, accAL_hbm, accBL_hbm), (uR_hbm, accAR_hbm, accBR_hbm)]):
        for last in (False, True):
            @pl.when((sid == 15) == last)
            def _():
                n = LAST_ROWS if last else W_STRIDE
                pltpu.sync_copy(u_half.at[pl.ds(base, n)],
                                u_sh.at[pl.ds(base, n)])
                for b in range(4):
                    pltpu.sync_copy(rows0_v,
                                    acc_sh.at[pl.ds(base + b * 128, 128)])
                pltpu.sync_copy(rows0_v.at[pl.ds(0, n - 512)],
                                acc_sh.at[pl.ds(base + 512, n - 512)])

        plsc.subcore_barrier()

        def blk_body(bb, _):
            pltpu.sync_copy(src_hbm.at[wid].at[pl.ds(bb * IB, IB)], src_v)
            pltpu.sync_copy(dst_hbm.at[wid].at[pl.ds(bb * IB, IB)], dst_v)
            hg = [None, None]
            hs = [None, None]
            pending = [False, False]
            hg[0] = pltpu.async_copy(u_sh.at[src_v.at[0]], rows[0], sgs[0])
            for c in range(IB):
                p = c % 2
                q = (c + 1) % 2
                hg[p].wait()
                hs[p] = pltpu.async_copy(
                    rows[p], acc_sh.at[dst_v.at[c]], sss[p], add=True)
                pending[p] = True
                if c + 1 < IB:
                    if pending[q]:
                        hs[q].wait()
                        pending[q] = False
                    hg[q] = pltpu.async_copy(
                        u_sh.at[src_v.at[c + 1]], rows[q], sgs[q])
            for q in range(2):
                if pending[q]:
                    hs[q].wait()
            return 0

        lax.fori_loop(0, CPW // IB, blk_body, 0)
        plsc.subcore_barrier()

        for last in (False, True):
            @pl.when((sid == 15) == last)
            def _():
                n = LAST_ROWS if last else W_STRIDE

                @pl.when(cid == 0)
                def _():
                    pltpu.sync_copy(acc_sh.at[pl.ds(base, n)],
                                    outA.at[pl.ds(base, n)])

                @pl.when(cid == 1)
                def _():
                    pltpu.sync_copy(acc_sh.at[pl.ds(base, n)],
                                    outB.at[pl.ds(base, n)])

        if h == 0:
            plsc.subcore_barrier()


def _row_spec(w):
    return pl.BlockSpec((BR, w), lambda i: (i, 0))


def _const_spec(hgt, w):
    return pl.BlockSpec((hgt, w), lambda i: (0, 0))


def _degmerge_body(a_ref, b_ref, o_ref):
    o_ref[...] = 1.0 / (a_ref[...] + b_ref[...] + 1.0)


def _degmerge(degA, degB):
    return pl.pallas_call(
        _degmerge_body,
        grid=(GRID,),
        in_specs=[_row_spec(16), _row_spec(16)],
        out_specs=_row_spec(16),
        out_shape=jax.ShapeDtypeStruct((NPAD, 16), f32),
    )(degA, degB)


def _mm_scale_body(h_ref, w_ref, b_ref, dg_ref, o_ref, oL_ref, oR_ref):
    acc = jnp.dot(h_ref[...], w_ref[...], preferred_element_type=f32)
    acc = acc + b_ref[...]
    u = jnp.sqrt(dg_ref[...][:, :1]) * acc
    o_ref[...] = u
    oL_ref[...] = u[:, :HH]
    oR_ref[...] = u[:, HH:]


def _mm_scale(h, W, b, dg):
    return pl.pallas_call(
        _mm_scale_body,
        grid=(GRID,),
        in_specs=[_row_spec(H), _const_spec(H, H), _const_spec(1, H), _row_spec(16)],
        out_specs=[_row_spec(H), _row_spec(HH), _row_spec(HH)],
        out_shape=[jax.ShapeDtypeStruct((NPAD, H), f32),
                   jax.ShapeDtypeStruct((NPAD, HH), f32),
                   jax.ShapeDtypeStruct((NPAD, HH), f32)],
    )(h, W, b.reshape(1, H), dg)


def _merge_body(aL_ref, aR_ref, bL_ref, bR_ref, u_ref, dg_ref,
                o_ref, oL_ref, oR_ref):
    a = jnp.concatenate([aL_ref[...], aR_ref[...]], axis=1)
    b = jnp.concatenate([bL_ref[...], bR_ref[...]], axis=1)
    un = dg_ref[...][:, :1] * (a + b + u_ref[...])
    o_ref[...] = un
    oL_ref[...] = un[:, :HH]
    oR_ref[...] = un[:, HH:]


def _merge(accAL, accAR, accBL, accBR, u, dg):
    return pl.pallas_call(
        _merge_body,
        grid=(GRID,),
        in_specs=[_row_spec(HH), _row_spec(HH), _row_spec(HH), _row_spec(HH),
                  _row_spec(H), _row_spec(16)],
        out_specs=[_row_spec(H), _row_spec(HH), _row_spec(HH)],
        out_shape=[jax.ShapeDtypeStruct((NPAD, H), f32),
                   jax.ShapeDtypeStruct((NPAD, HH), f32),
                   jax.ShapeDtypeStruct((NPAD, HH), f32)],
    )(accAL, accAR, accBL, accBR, u, dg)


def _combine_body(u0, u1, u2, u3, u4, s_ref, dg_ref, o_ref):
    s = s_ref[...]
    agg = (u0[...] * s[0:1, :] + u1[...] * s[1:2, :] + u2[...] * s[2:3, :]
           + u3[...] * s[3:4, :] + u4[...] * s[4:5, :])
    o_ref[...] = jnp.maximum(lax.rsqrt(dg_ref[...][:, :1]) * agg, 0.0)


def _combine(us, satt, dg):
    return pl.pallas_call(
        _combine_body,
        grid=(GRID,),
        in_specs=[_row_spec(H)] * 5 + [_const_spec(8, H), _row_spec(16)],
        out_specs=_row_spec(H),
        out_shape=jax.ShapeDtypeStruct((NPAD, H), f32),
    )(*us, satt, dg)


def _out_body(h_ref, w_ref, b_ref, o_ref):
    logits = jnp.dot(h_ref[...], w_ref[...], preferred_element_type=f32)
    logits = logits + b_ref[...]
    m = jnp.max(logits, axis=1, keepdims=True)
    e = jnp.exp(logits - m)
    s = jnp.sum(e, axis=1, keepdims=True)
    o_ref[...] = logits - m - jnp.log(s)


def _out(h, Wp, bp):
    return pl.pallas_call(
        _out_body,
        grid=(GRID,),
        in_specs=[_row_spec(H), _const_spec(H, H), _const_spec(1, H)],
        out_specs=_row_spec(H),
        out_shape=jax.ShapeDtypeStruct((NPAD, H), f32),
    )(h, Wp, bp)


def kernel(x, edge_index, W0, b0, W1, b1, Wout, bout, att, alphas):
    row0 = edge_index[0]
    col0 = edge_index[1]
    dsts = jnp.concatenate([row0, col0])
    srcs = jnp.concatenate([col0, row0])
    pad_e = EPAD - E2
    dst3 = jnp.concatenate(
        [dsts, jnp.full((pad_e,), TRASH, i32)]).reshape(NW, CPW, CHUNK)
    src3 = jnp.concatenate(
        [srcs, jnp.zeros((pad_e,), i32)]).reshape(NW, CPW, CHUNK)

    xp = jnp.zeros((NPAD, D), f32).at[:N].set(x)
    Wp = jnp.zeros((H, 128), f32).at[:, :OUT].set(Wout)
    bp = jnp.full((128,), -1e30, f32).at[:OUT].set(bout).reshape(1, 128)

    al = [ALPHA * jnp.tanh(alphas[j]) for j in range(K + 1)]
    satts = []
    for i in range(NLAYER):
        s = _satt(att[i], al)
        satts.append(jnp.zeros((8, H), f32).at[:K + 1].set(s))

    degA, degB = _deg_k(dst3)
    dg = _degmerge(degA, degB)

    h = xp
    Ws = [(W0, b0), (W1, b1)]
    for i in range(NLAYER):
        Wi, bi = Ws[i]
        u, uL, uR = _mm_scale(h, Wi, bi, dg)
        us = [u]
        for _ in range(K):
            aAL, aAR, aBL, aBR = _prop_k(uL, uR, src3, dst3)
            u, uL, uR = _merge(aAL, aAR, aBL, aBR, u, dg)
            us.append(u)
        h = _combine(us, satts[i], dg)

    res = _out(h, Wp, bp)
    return res[:N, :OUT]


# speedup vs baseline: 5.9710x; 1.0064x over previous
import functools

import jax
import jax.numpy as jnp
from jax import lax
from jax.experimental import pallas as pl
from jax.experimental.pallas import tpu as pltpu
from jax.experimental.pallas import tpu_sc as plsc

N = 10000
D = 128
H = 128
K = 4
NLAYER = 2
OUT = 10
ALPHA = 1.0
PA = 1.0
PB = 1.0

NPAD = 10240
ACC_ROWS = 10016
E2 = 2 * 320000
NW = 32
CHUNK = 128
IB = 32
CPW = -(-E2 // (NW * CHUNK * IB)) * IB
EPAD = NW * CPW * CHUNK
TRASH = ACC_ROWS - 1
W_STRIDE = 632
LAST_ROWS = ACC_ROWS - 15 * W_STRIDE

BR = 1024
GRID = NPAD // BR

f32 = jnp.float32
i32 = jnp.int32


def _shift1(v):
    return jnp.concatenate([jnp.zeros((1,), v.dtype), v[:-1]])


def _expansion(prev, L, al, a, b, l=-1.0, r=1.0):
    if L == 0:
        return prev[0]
    if L == 1:
        coef1 = ((a - b) / 2.0 - (a + b + 2.0) / 2.0 * (l + r) / (r - l)) * al[0]
        coef2 = ((a + b + 2.0) / (r - l)) * al[0]
        t1 = prev[-1]
        return coef1 * t1 + coef2 * _shift1(t1)
    coef_l = 2.0 * L * (L + a + b) * (2.0 * L - 2.0 + a + b)
    coef_lm1_1 = (2.0 * L + a + b - 1.0) * (2.0 * L + a + b) * (2.0 * L + a + b - 2.0)
    coef_lm1_2 = (2.0 * L + a + b - 1.0) * (a ** 2 - b ** 2)
    coef_lm2 = 2.0 * (L - 1.0 + a) * (L - 1.0 + b) * (2.0 * L + a + b)
    tmp1 = al[L - 1] * (coef_lm1_1 / coef_l)
    tmp2 = al[L - 1] * (coef_lm1_2 / coef_l)
    tmp3 = al[L - 1] * al[L - 2] * (coef_lm2 / coef_l)
    tmp1_2 = tmp1 * (2.0 / (r - l))
    tmp2_2 = tmp1 * ((r + l) / (r - l)) + tmp2
    t1 = prev[-1]
    return tmp1_2 * _shift1(t1) - tmp2_2 * t1 - tmp3 * prev[-2]


def _satt(att_i, al):
    tmp0 = jnp.zeros((K + 1,), f32).at[0].set(1.0)
    xs = [tmp0]
    s = att_i[:, 0][:, None] * tmp0[None, :]
    for j in range(1, K + 1):
        tx = _expansion(xs, j, al, PA, PB)
        s = s + att_i[:, j][:, None] * tx[None, :]
        xs.append(tx)
    return s.T


_MESH = plsc.VectorSubcoreMesh(core_axis_name="c", subcore_axis_name="s")


@functools.partial(
    pl.kernel,
    mesh=_MESH,
    out_type=[jax.ShapeDtypeStruct((NPAD, 16), f32),
              jax.ShapeDtypeStruct((NPAD, 16), f32)],
    scratch_types=[
        pltpu.VMEM_SHARED((ACC_ROWS, 16), f32),
        pltpu.VMEM((IB, CHUNK), i32),
        pltpu.VMEM((CHUNK, 16), f32),
        pltpu.VMEM((CHUNK, 16), f32),
        pltpu.SemaphoreType.DMA,
    ],
)
def _deg_k(dst_hbm, degA_hbm, degB_hbm, acc_sh, dst_v, ones_v, zeros_v, sd):
    cid = lax.axis_index("c")
    sid = lax.axis_index("s")
    wid = sid * 2 + cid
    base = sid * W_STRIDE

    def fill(rr, _):
        ones_v[rr] = jnp.full((16,), 1.0, f32)
        zeros_v[rr] = jnp.zeros((16,), f32)
        return 0

    lax.fori_loop(0, CHUNK, fill, 0)
    for b in range(4):
        pltpu.sync_copy(zeros_v, acc_sh.at[pl.ds(base + b * 128, 128)])

    @pl.when(sid < 15)
    def _():
        pltpu.sync_copy(zeros_v.at[pl.ds(0, W_STRIDE - 512)],
                        acc_sh.at[pl.ds(base + 512, W_STRIDE - 512)])

    @pl.when(sid == 15)
    def _():
        pltpu.sync_copy(zeros_v.at[pl.ds(0, LAST_ROWS - 512)],
                        acc_sh.at[pl.ds(base + 512, LAST_ROWS - 512)])

    plsc.subcore_barrier()

    def blk_body(bb, _):
        pltpu.sync_copy(dst_hbm.at[wid].at[pl.ds(bb * IB, IB)], dst_v)
        hh = [pltpu.async_copy(ones_v, acc_sh.at[dst_v.at[c]], sd, add=True)
              for c in range(IB)]
        for handle in hh:
            handle.wait()
        return 0

    lax.fori_loop(0, CPW // IB, blk_body, 0)
    plsc.subcore_barrier()

    for last in (False, True):
        @pl.when((sid == 15) == last)
        def _():
            n = LAST_ROWS if last else W_STRIDE

            @pl.when(cid == 0)
            def _():
                pltpu.sync_copy(acc_sh.at[pl.ds(base, n)],
                                degA_hbm.at[pl.ds(base, n)])

            @pl.when(cid == 1)
            def _():
                pltpu.sync_copy(acc_sh.at[pl.ds(base, n)],
                                degB_hbm.at[pl.ds(base, n)])


HH = H // 2


@functools.partial(
    pl.kernel,
    mesh=_MESH,
    out_type=[jax.ShapeDtypeStruct((NPAD, HH), f32),
              jax.ShapeDtypeStruct((NPAD, HH), f32),
              jax.ShapeDtypeStruct((NPAD, HH), f32),
              jax.ShapeDtypeStruct((NPAD, HH), f32)],
    scratch_types=[
        pltpu.VMEM_SHARED((ACC_ROWS, HH), f32),
        pltpu.VMEM_SHARED((ACC_ROWS, HH), f32),
        pltpu.VMEM((IB, CHUNK), i32),
        pltpu.VMEM((IB, CHUNK), i32),
        pltpu.VMEM((CHUNK, HH), f32),
        pltpu.VMEM((CHUNK, HH), f32),
        pltpu.SemaphoreType.DMA,
        pltpu.SemaphoreType.DMA,
        pltpu.SemaphoreType.DMA,
        pltpu.SemaphoreType.DMA,
    ],
)
def _prop_k(uL_hbm, uR_hbm, src_hbm, dst_hbm,
            accAL_hbm, accAR_hbm, accBL_hbm, accBR_hbm,
            u_sh, acc_sh, src_v, dst_v, rows0_v, rows1_v,
            sg0, sg1, ss0, ss1):
    cid = lax.axis_index("c")
    sid = lax.axis_index("s")
    wid = sid * 2 + cid
    base = sid * W_STRIDE
    zeros16 = jnp.zeros((16,), f32)
    rows = [rows0_v, rows1_v]
    sgs = [sg0, sg1]
    sss = [ss0, ss1]

    def zrow(rr, _):
        for k in range(HH // 16):
            rows0_v[rr, k * 16:(k + 1) * 16] = zeros16
        return 0

    lax.fori_loop(0, CHUNK, zrow, 0)

    for h, (u_half, outA, outB) in enumerate(
            [(uL_hbm, accAL_hbm, accBL_hbm), (uR_hbm, accAR_hbm, accBR_hbm)]):
        for last in (False, True):
            @pl.when((sid == 15) == last)
            def _():
                n = LAST_ROWS if last else W_STRIDE
                pltpu.sync_copy(u_half.at[pl.ds(base, n)],
                                u_sh.at[pl.ds(base, n)])
                for b in range(4):
                    pltpu.sync_copy(rows0_v,
                                    acc_sh.at[pl.ds(base + b * 128, 128)])
                pltpu.sync_copy(rows0_v.at[pl.ds(0, n - 512)],
                                acc_sh.at[pl.ds(base + 512, n - 512)])

        plsc.subcore_barrier()

        def blk_body(bb, _):
            pltpu.sync_copy(src_hbm.at[wid].at[pl.ds(bb * IB, IB)], src_v)
            pltpu.sync_copy(dst_hbm.at[wid].at[pl.ds(bb * IB, IB)], dst_v)
            hg = [None, None]
            hs = [None, None]
            pending = [False, False]
            hg[0] = pltpu.async_copy(u_sh.at[src_v.at[0]], rows[0], sgs[0])
            for c in range(IB):
                p = c % 2
                q = (c + 1) % 2
                hg[p].wait()
                hs[p] = pltpu.async_copy(
                    rows[p], acc_sh.at[dst_v.at[c]], sss[p], add=True)
                pending[p] = True
                if c + 1 < IB:
                    if pending[q]:
                        hs[q].wait()
                        pending[q] = False
                    hg[q] = pltpu.async_copy(
                        u_sh.at[src_v.at[c + 1]], rows[q], sgs[q])
            for q in range(2):
                if pending[q]:
                    hs[q].wait()
            return 0

        lax.fori_loop(0, CPW // IB, blk_body, 0)
        plsc.subcore_barrier()

        for last in (False, True):
            @pl.when((sid == 15) == last)
            def _():
                n = LAST_ROWS if last else W_STRIDE

                @pl.when(cid == 0)
                def _():
                    pltpu.sync_copy(acc_sh.at[pl.ds(base, n)],
                                    outA.at[pl.ds(base, n)])

                @pl.when(cid == 1)
                def _():
                    pltpu.sync_copy(acc_sh.at[pl.ds(base, n)],
                                    outB.at[pl.ds(base, n)])

        if h == 0:
            plsc.subcore_barrier()


def _row_spec(w):
    return pl.BlockSpec((BR, w), lambda i: (i, 0))


def _const_spec(hgt, w):
    return pl.BlockSpec((hgt, w), lambda i: (0, 0))


def _degmerge_body(a_ref, b_ref, o_ref):
    o_ref[...] = 1.0 / (a_ref[...] + b_ref[...] + 1.0)


def _degmerge(degA, degB):
    return pl.pallas_call(
        _degmerge_body,
        grid=(GRID,),
        in_specs=[_row_spec(16), _row_spec(16)],
        out_specs=_row_spec(16),
        out_shape=jax.ShapeDtypeStruct((NPAD, 16), f32),
    )(degA, degB)


def _mm_scale_body(h_ref, w_ref, b_ref, dg_ref, o_ref, oL_ref, oR_ref):
    acc = jnp.dot(h_ref[...], w_ref[...], preferred_element_type=f32)
    acc = acc + b_ref[...]
    u = jnp.sqrt(dg_ref[...][:, :1]) * acc
    o_ref[...] = u
    oL_ref[...] = u[:, :HH]
    oR_ref[...] = u[:, HH:]


def _mm_scale(h, W, b, dg):
    return pl.pallas_call(
        _mm_scale_body,
        grid=(GRID,),
        in_specs=[_row_spec(H), _const_spec(H, H), _const_spec(1, H), _row_spec(16)],
        out_specs=[_row_spec(H), _row_spec(HH), _row_spec(HH)],
        out_shape=[jax.ShapeDtypeStruct((NPAD, H), f32),
                   jax.ShapeDtypeStruct((NPAD, HH), f32),
                   jax.ShapeDtypeStruct((NPAD, HH), f32)],
    )(h, W, b.reshape(1, H), dg)


def _merge_body(aL_ref, aR_ref, bL_ref, bR_ref, u_ref, dg_ref,
                o_ref, oL_ref, oR_ref):
    a = jnp.concatenate([aL_ref[...], aR_ref[...]], axis=1)
    b = jnp.concatenate([bL_ref[...], bR_ref[...]], axis=1)
    un = dg_ref[...][:, :1] * (a + b + u_ref[...])
    o_ref[...] = un
    oL_ref[...] = un[:, :HH]
    oR_ref[...] = un[:, HH:]


def _merge(accAL, accAR, accBL, accBR, u, dg):
    return pl.pallas_call(
        _merge_body,
        grid=(GRID,),
        in_specs=[_row_spec(HH), _row_spec(HH), _row_spec(HH), _row_spec(HH),
                  _row_spec(H), _row_spec(16)],
        out_specs=[_row_spec(H), _row_spec(HH), _row_spec(HH)],
        out_shape=[jax.ShapeDtypeStruct((NPAD, H), f32),
                   jax.ShapeDtypeStruct((NPAD, HH), f32),
                   jax.ShapeDtypeStruct((NPAD, HH), f32)],
    )(accAL, accAR, accBL, accBR, u, dg)


def _agg_relu(u0, u1, u2, u3, u4, s, dg):
    agg = (u0[...] * s[0:1, :] + u1[...] * s[1:2, :] + u2[...] * s[2:3, :]
           + u3[...] * s[3:4, :] + u4[...] * s[4:5, :])
    return jnp.maximum(lax.rsqrt(dg[:, :1]) * agg, 0.0)


def _combine_mm_body(u0, u1, u2, u3, u4, s_ref, dg_ref, w_ref, b_ref,
                     o_ref, oL_ref, oR_ref):
    hv = _agg_relu(u0, u1, u2, u3, u4, s_ref[...], dg_ref[...])
    acc = jnp.dot(hv, w_ref[...], preferred_element_type=f32) + b_ref[...]
    u = jnp.sqrt(dg_ref[...][:, :1]) * acc
    o_ref[...] = u
    oL_ref[...] = u[:, :HH]
    oR_ref[...] = u[:, HH:]


def _combine_mm(us, satt, dg, W, b):
    return pl.pallas_call(
        _combine_mm_body,
        grid=(GRID,),
        in_specs=[_row_spec(H)] * 5 + [_const_spec(8, H), _row_spec(16),
                                       _const_spec(H, H), _const_spec(1, H)],
        out_specs=[_row_spec(H), _row_spec(HH), _row_spec(HH)],
        out_shape=[jax.ShapeDtypeStruct((NPAD, H), f32),
                   jax.ShapeDtypeStruct((NPAD, HH), f32),
                   jax.ShapeDtypeStruct((NPAD, HH), f32)],
    )(*us, satt, dg, W, b.reshape(1, H))


def _combine_out_body(u0, u1, u2, u3, u4, s_ref, dg_ref, w_ref, b_ref, o_ref):
    hv = _agg_relu(u0, u1, u2, u3, u4, s_ref[...], dg_ref[...])
    logits = jnp.dot(hv, w_ref[...], preferred_element_type=f32) + b_ref[...]
    m = jnp.max(logits, axis=1, keepdims=True)
    e = jnp.exp(logits - m)
    sm = jnp.sum(e, axis=1, keepdims=True)
    o_ref[...] = logits - m - jnp.log(sm)


def _combine_out(us, satt, dg, Wp, bp):
    return pl.pallas_call(
        _combine_out_body,
        grid=(GRID,),
        in_specs=[_row_spec(H)] * 5 + [_const_spec(8, H), _row_spec(16),
                                       _const_spec(H, H), _const_spec(1, H)],
        out_specs=_row_spec(H),
        out_shape=jax.ShapeDtypeStruct((NPAD, H), f32),
    )(*us, satt, dg, Wp, bp)


def kernel(x, edge_index, W0, b0, W1, b1, Wout, bout, att, alphas):
    row0 = edge_index[0]
    col0 = edge_index[1]
    dsts = jnp.concatenate([row0, col0])
    srcs = jnp.concatenate([col0, row0])
    pad_e = EPAD - E2
    dst3 = jnp.concatenate(
        [dsts, jnp.full((pad_e,), TRASH, i32)]).reshape(NW, CPW, CHUNK)
    src3 = jnp.concatenate(
        [srcs, jnp.zeros((pad_e,), i32)]).reshape(NW, CPW, CHUNK)

    xp = jnp.zeros((NPAD, D), f32).at[:N].set(x)
    Wp = jnp.zeros((H, 128), f32).at[:, :OUT].set(Wout)
    bp = jnp.full((128,), -1e30, f32).at[:OUT].set(bout).reshape(1, 128)

    al = [ALPHA * jnp.tanh(alphas[j]) for j in range(K + 1)]
    satts = []
    for i in range(NLAYER):
        s = _satt(att[i], al)
        satts.append(jnp.zeros((8, H), f32).at[:K + 1].set(s))

    degA, degB = _deg_k(dst3)
    dg = _degmerge(degA, degB)

    def props(uL, uR, u):
        us = [u]
        for _ in range(K):
            aAL, aAR, aBL, aBR = _prop_k(uL, uR, src3, dst3)
            u, uL, uR = _merge(aAL, aAR, aBL, aBR, u, dg)
            us.append(u)
        return us

    u, uL, uR = _mm_scale(xp, W0, b0, dg)
    us = props(uL, uR, u)
    u, uL, uR = _combine_mm(us, satts[0], dg, W1, b1)
    us = props(uL, uR, u)
    res = _combine_out(us, satts[1], dg, Wp, bp)
    return res[:N, :OUT]


# speedup vs baseline: 6.0683x; 1.0163x over previous
import functools

import jax
import jax.numpy as jnp
from jax import lax
from jax.experimental import pallas as pl
from jax.experimental.pallas import tpu as pltpu
from jax.experimental.pallas import tpu_sc as plsc

N = 10000
D = 128
H = 128
K = 4
NLAYER = 2
OUT = 10
ALPHA = 1.0
PA = 1.0
PB = 1.0

NPAD = 10240
ACC_ROWS = 10016
E2 = 2 * 320000
NW = 32
CHUNK = 128
IB = 16
CPW = -(-E2 // (NW * CHUNK * IB)) * IB
EPAD = NW * CPW * CHUNK
TRASH = ACC_ROWS - 1
W_STRIDE = 632
LAST_ROWS = ACC_ROWS - 15 * W_STRIDE

BR = 1024
GRID = NPAD // BR

f32 = jnp.float32
i32 = jnp.int32


def _shift1(v):
    return jnp.concatenate([jnp.zeros((1,), v.dtype), v[:-1]])


def _expansion(prev, L, al, a, b, l=-1.0, r=1.0):
    if L == 0:
        return prev[0]
    if L == 1:
        coef1 = ((a - b) / 2.0 - (a + b + 2.0) / 2.0 * (l + r) / (r - l)) * al[0]
        coef2 = ((a + b + 2.0) / (r - l)) * al[0]
        t1 = prev[-1]
        return coef1 * t1 + coef2 * _shift1(t1)
    coef_l = 2.0 * L * (L + a + b) * (2.0 * L - 2.0 + a + b)
    coef_lm1_1 = (2.0 * L + a + b - 1.0) * (2.0 * L + a + b) * (2.0 * L + a + b - 2.0)
    coef_lm1_2 = (2.0 * L + a + b - 1.0) * (a ** 2 - b ** 2)
    coef_lm2 = 2.0 * (L - 1.0 + a) * (L - 1.0 + b) * (2.0 * L + a + b)
    tmp1 = al[L - 1] * (coef_lm1_1 / coef_l)
    tmp2 = al[L - 1] * (coef_lm1_2 / coef_l)
    tmp3 = al[L - 1] * al[L - 2] * (coef_lm2 / coef_l)
    tmp1_2 = tmp1 * (2.0 / (r - l))
    tmp2_2 = tmp1 * ((r + l) / (r - l)) + tmp2
    t1 = prev[-1]
    return tmp1_2 * _shift1(t1) - tmp2_2 * t1 - tmp3 * prev[-2]


def _satt(att_i, al):
    tmp0 = jnp.zeros((K + 1,), f32).at[0].set(1.0)
    xs = [tmp0]
    s = att_i[:, 0][:, None] * tmp0[None, :]
    for j in range(1, K + 1):
        tx = _expansion(xs, j, al, PA, PB)
        s = s + att_i[:, j][:, None] * tx[None, :]
        xs.append(tx)
    return s.T


_MESH = plsc.VectorSubcoreMesh(core_axis_name="c", subcore_axis_name="s")


@functools.partial(
    pl.kernel,
    mesh=_MESH,
    out_type=[jax.ShapeDtypeStruct((NPAD, 16), f32),
              jax.ShapeDtypeStruct((NPAD, 16), f32)],
    scratch_types=[
        pltpu.VMEM_SHARED((ACC_ROWS, 16), f32),
        pltpu.VMEM((IB, CHUNK), i32),
        pltpu.VMEM((CHUNK, 16), f32),
        pltpu.VMEM((CHUNK, 16), f32),
        pltpu.SemaphoreType.DMA,
    ],
)
def _deg_k(dst_hbm, degA_hbm, degB_hbm, acc_sh, dst_v, ones_v, zeros_v, sd):
    cid = lax.axis_index("c")
    sid = lax.axis_index("s")
    wid = sid * 2 + cid
    base = sid * W_STRIDE

    def fill(rr, _):
        ones_v[rr] = jnp.full((16,), 1.0, f32)
        zeros_v[rr] = jnp.zeros((16,), f32)
        return 0

    lax.fori_loop(0, CHUNK, fill, 0)
    for b in range(4):
        pltpu.sync_copy(zeros_v, acc_sh.at[pl.ds(base + b * 128, 128)])

    @pl.when(sid < 15)
    def _():
        pltpu.sync_copy(zeros_v.at[pl.ds(0, W_STRIDE - 512)],
                        acc_sh.at[pl.ds(base + 512, W_STRIDE - 512)])

    @pl.when(sid == 15)
    def _():
        pltpu.sync_copy(zeros_v.at[pl.ds(0, LAST_ROWS - 512)],
                        acc_sh.at[pl.ds(base + 512, LAST_ROWS - 512)])

    plsc.subcore_barrier()

    def blk_body(bb, _):
        pltpu.sync_copy(dst_hbm.at[wid].at[pl.ds(bb * IB, IB)], dst_v)
        hh = [pltpu.async_copy(ones_v, acc_sh.at[dst_v.at[c]], sd, add=True)
              for c in range(IB)]
        for handle in hh:
            handle.wait()
        return 0

    lax.fori_loop(0, CPW // IB, blk_body, 0)
    plsc.subcore_barrier()

    for last in (False, True):
        @pl.when((sid == 15) == last)
        def _():
            n = LAST_ROWS if last else W_STRIDE

            @pl.when(cid == 0)
            def _():
                pltpu.sync_copy(acc_sh.at[pl.ds(base, n)],
                                degA_hbm.at[pl.ds(base, n)])

            @pl.when(cid == 1)
            def _():
                pltpu.sync_copy(acc_sh.at[pl.ds(base, n)],
                                degB_hbm.at[pl.ds(base, n)])


HH = H // 2


@functools.partial(
    pl.kernel,
    mesh=_MESH,
    out_type=[jax.ShapeDtypeStruct((NPAD, HH), f32),
              jax.ShapeDtypeStruct((NPAD, HH), f32),
              jax.ShapeDtypeStruct((NPAD, HH), f32),
              jax.ShapeDtypeStruct((NPAD, HH), f32)],
    scratch_types=[
        pltpu.VMEM_SHARED((ACC_ROWS, HH), f32),
        pltpu.VMEM_SHARED((ACC_ROWS, HH), f32),
        pltpu.VMEM((IB, CHUNK), i32),
        pltpu.VMEM((IB, CHUNK), i32),
        pltpu.VMEM((IB, CHUNK), i32),
        pltpu.VMEM((IB, CHUNK), i32),
        pltpu.VMEM((CHUNK, HH), f32),
        pltpu.VMEM((CHUNK, HH), f32),
        pltpu.SemaphoreType.DMA,
        pltpu.SemaphoreType.DMA,
        pltpu.SemaphoreType.DMA,
        pltpu.SemaphoreType.DMA,
        pltpu.SemaphoreType.DMA,
        pltpu.SemaphoreType.DMA,
    ],
)
def _prop_k(uL_hbm, uR_hbm, src_hbm, dst_hbm,
            accAL_hbm, accAR_hbm, accBL_hbm, accBR_hbm,
            u_sh, acc_sh, src_v0, dst_v0, src_v1, dst_v1, rows0_v, rows1_v,
            sg0, sg1, ss0, ss1, si0, si1):
    cid = lax.axis_index("c")
    sid = lax.axis_index("s")
    wid = sid * 2 + cid
    base = sid * W_STRIDE
    zeros16 = jnp.zeros((16,), f32)
    rows = [rows0_v, rows1_v]
    sgs = [sg0, sg1]
    sss = [ss0, ss1]

    def zrow(rr, _):
        for k in range(HH // 16):
            rows0_v[rr, k * 16:(k + 1) * 16] = zeros16
        return 0

    lax.fori_loop(0, CHUNK, zrow, 0)

    for h, (u_half, outA, outB) in enumerate(
            [(uL_hbm, accAL_hbm, accBL_hbm), (uR_hbm, accAR_hbm, accBR_hbm)]):
        for last in (False, True):
            @pl.when((sid == 15) == last)
            def _():
                n = LAST_ROWS if last else W_STRIDE
                pltpu.sync_copy(u_half.at[pl.ds(base, n)],
                                u_sh.at[pl.ds(base, n)])
                for b in range(4):
                    pltpu.sync_copy(rows0_v,
                                    acc_sh.at[pl.ds(base + b * 128, 128)])
                pltpu.sync_copy(rows0_v.at[pl.ds(0, n - 512)],
                                acc_sh.at[pl.ds(base + 512, n - 512)])

        plsc.subcore_barrier()

        def chunk_loop(src_v, dst_v):
            hg = [None, None]
            hs = [None, None]
            pending = [False, False]
            hg[0] = pltpu.async_copy(u_sh.at[src_v.at[0]], rows[0], sgs[0])
            for c in range(IB):
                p = c % 2
                q = (c + 1) % 2
                hg[p].wait()
                hs[p] = pltpu.async_copy(
                    rows[p], acc_sh.at[dst_v.at[c]], sss[p], add=True)
                pending[p] = True
                if c + 1 < IB:
                    if pending[q]:
                        hs[q].wait()
                        pending[q] = False
                    hg[q] = pltpu.async_copy(
                        u_sh.at[src_v.at[c + 1]], rows[q], sgs[q])
            for q in range(2):
                if pending[q]:
                    hs[q].wait()

        npairs = (CPW // IB) // 2
        sblk = src_hbm.at[wid]
        dblk = dst_hbm.at[wid]

        def fetch(blk_id, sv, dv, sem):
            pltpu.async_copy(sblk.at[pl.ds(blk_id * IB, IB)], sv, sem)
            pltpu.async_copy(dblk.at[pl.ds(blk_id * IB, IB)], dv, sem)

        def drain_idx(sv, dv, sem):
            pltpu.make_async_copy(sblk.at[pl.ds(0, IB)], sv, sem).wait()
            pltpu.make_async_copy(dblk.at[pl.ds(0, IB)], dv, sem).wait()

        fetch(0, src_v0, dst_v0, si0)
        fetch(1, src_v1, dst_v1, si1)

        def pair_body(bb, _):
            drain_idx(src_v0, dst_v0, si0)
            chunk_loop(src_v0, dst_v0)

            @pl.when(bb + 1 < npairs)
            def _():
                fetch(2 * bb + 2, src_v0, dst_v0, si0)

            drain_idx(src_v1, dst_v1, si1)
            chunk_loop(src_v1, dst_v1)

            @pl.when(bb + 1 < npairs)
            def _():
                fetch(2 * bb + 3, src_v1, dst_v1, si1)

            return 0

        lax.fori_loop(0, npairs, pair_body, 0)
        plsc.subcore_barrier()

        for last in (False, True):
            @pl.when((sid == 15) == last)
            def _():
                n = LAST_ROWS if last else W_STRIDE

                @pl.when(cid == 0)
                def _():
                    pltpu.sync_copy(acc_sh.at[pl.ds(base, n)],
                                    outA.at[pl.ds(base, n)])

                @pl.when(cid == 1)
                def _():
                    pltpu.sync_copy(acc_sh.at[pl.ds(base, n)],
                                    outB.at[pl.ds(base, n)])

        if h == 0:
            plsc.subcore_barrier()


def _row_spec(w):
    return pl.BlockSpec((BR, w), lambda i: (i, 0))


def _const_spec(hgt, w):
    return pl.BlockSpec((hgt, w), lambda i: (0, 0))


def _degmerge_body(a_ref, b_ref, o_ref):
    o_ref[...] = 1.0 / (a_ref[...] + b_ref[...] + 1.0)


def _degmerge(degA, degB):
    return pl.pallas_call(
        _degmerge_body,
        grid=(GRID,),
        in_specs=[_row_spec(16), _row_spec(16)],
        out_specs=_row_spec(16),
        out_shape=jax.ShapeDtypeStruct((NPAD, 16), f32),
    )(degA, degB)


def _mm_scale_body(h_ref, w_ref, b_ref, dg_ref, o_ref, oL_ref, oR_ref):
    acc = jnp.dot(h_ref[...], w_ref[...], preferred_element_type=f32)
    acc = acc + b_ref[...]
    u = jnp.sqrt(dg_ref[...][:, :1]) * acc
    o_ref[...] = u
    oL_ref[...] = u[:, :HH]
    oR_ref[...] = u[:, HH:]


def _mm_scale(h, W, b, dg):
    return pl.pallas_call(
        _mm_scale_body,
        grid=(GRID,),
        in_specs=[_row_spec(H), _const_spec(H, H), _const_spec(1, H), _row_spec(16)],
        out_specs=[_row_spec(H), _row_spec(HH), _row_spec(HH)],
        out_shape=[jax.ShapeDtypeStruct((NPAD, H), f32),
                   jax.ShapeDtypeStruct((NPAD, HH), f32),
                   jax.ShapeDtypeStruct((NPAD, HH), f32)],
    )(h, W, b.reshape(1, H), dg)


def _merge_body(aL_ref, aR_ref, bL_ref, bR_ref, u_ref, dg_ref,
                o_ref, oL_ref, oR_ref):
    a = jnp.concatenate([aL_ref[...], aR_ref[...]], axis=1)
    b = jnp.concatenate([bL_ref[...], bR_ref[...]], axis=1)
    un = dg_ref[...][:, :1] * (a + b + u_ref[...])
    o_ref[...] = un
    oL_ref[...] = un[:, :HH]
    oR_ref[...] = un[:, HH:]


def _merge(accAL, accAR, accBL, accBR, u, dg):
    return pl.pallas_call(
        _merge_body,
        grid=(GRID,),
        in_specs=[_row_spec(HH), _row_spec(HH), _row_spec(HH), _row_spec(HH),
                  _row_spec(H), _row_spec(16)],
        out_specs=[_row_spec(H), _row_spec(HH), _row_spec(HH)],
        out_shape=[jax.ShapeDtypeStruct((NPAD, H), f32),
                   jax.ShapeDtypeStruct((NPAD, HH), f32),
                   jax.ShapeDtypeStruct((NPAD, HH), f32)],
    )(accAL, accAR, accBL, accBR, u, dg)


def _agg_relu(u0, u1, u2, u3, u4, s, dg):
    agg = (u0[...] * s[0:1, :] + u1[...] * s[1:2, :] + u2[...] * s[2:3, :]
           + u3[...] * s[3:4, :] + u4[...] * s[4:5, :])
    return jnp.maximum(lax.rsqrt(dg[:, :1]) * agg, 0.0)


def _combine_mm_body(u0, u1, u2, u3, u4, s_ref, dg_ref, w_ref, b_ref,
                     o_ref, oL_ref, oR_ref):
    hv = _agg_relu(u0, u1, u2, u3, u4, s_ref[...], dg_ref[...])
    acc = jnp.dot(hv, w_ref[...], preferred_element_type=f32) + b_ref[...]
    u = jnp.sqrt(dg_ref[...][:, :1]) * acc
    o_ref[...] = u
    oL_ref[...] = u[:, :HH]
    oR_ref[...] = u[:, HH:]


def _combine_mm(us, satt, dg, W, b):
    return pl.pallas_call(
        _combine_mm_body,
        grid=(GRID,),
        in_specs=[_row_spec(H)] * 5 + [_const_spec(8, H), _row_spec(16),
                                       _const_spec(H, H), _const_spec(1, H)],
        out_specs=[_row_spec(H), _row_spec(HH), _row_spec(HH)],
        out_shape=[jax.ShapeDtypeStruct((NPAD, H), f32),
                   jax.ShapeDtypeStruct((NPAD, HH), f32),
                   jax.ShapeDtypeStruct((NPAD, HH), f32)],
    )(*us, satt, dg, W, b.reshape(1, H))


def _combine_out_body(u0, u1, u2, u3, u4, s_ref, dg_ref, w_ref, b_ref, o_ref):
    hv = _agg_relu(u0, u1, u2, u3, u4, s_ref[...], dg_ref[...])
    logits = jnp.dot(hv, w_ref[...], preferred_element_type=f32) + b_ref[...]
    m = jnp.max(logits, axis=1, keepdims=True)
    e = jnp.exp(logits - m)
    sm = jnp.sum(e, axis=1, keepdims=True)
    o_ref[...] = logits - m - jnp.log(sm)


def _combine_out(us, satt, dg, Wp, bp):
    return pl.pallas_call(
        _combine_out_body,
        grid=(GRID,),
        in_specs=[_row_spec(H)] * 5 + [_const_spec(8, H), _row_spec(16),
                                       _const_spec(H, H), _const_spec(1, H)],
        out_specs=_row_spec(H),
        out_shape=jax.ShapeDtypeStruct((NPAD, H), f32),
    )(*us, satt, dg, Wp, bp)


def kernel(x, edge_index, W0, b0, W1, b1, Wout, bout, att, alphas):
    row0 = edge_index[0]
    col0 = edge_index[1]
    dsts = jnp.concatenate([row0, col0])
    srcs = jnp.concatenate([col0, row0])
    pad_e = EPAD - E2
    dst3 = jnp.concatenate(
        [dsts, jnp.full((pad_e,), TRASH, i32)]).reshape(NW, CPW, CHUNK)
    src3 = jnp.concatenate(
        [srcs, jnp.zeros((pad_e,), i32)]).reshape(NW, CPW, CHUNK)

    xp = jnp.zeros((NPAD, D), f32).at[:N].set(x)
    Wp = jnp.zeros((H, 128), f32).at[:, :OUT].set(Wout)
    bp = jnp.full((128,), -1e30, f32).at[:OUT].set(bout).reshape(1, 128)

    al = [ALPHA * jnp.tanh(alphas[j]) for j in range(K + 1)]
    satts = []
    for i in range(NLAYER):
        s = _satt(att[i], al)
        satts.append(jnp.zeros((8, H), f32).at[:K + 1].set(s))

    degA, degB = _deg_k(dst3)
    dg = _degmerge(degA, degB)

    def props(uL, uR, u):
        us = [u]
        for _ in range(K):
            aAL, aAR, aBL, aBR = _prop_k(uL, uR, src3, dst3)
            u, uL, uR = _merge(aAL, aAR, aBL, aBR, u, dg)
            us.append(u)
        return us

    u, uL, uR = _mm_scale(xp, W0, b0, dg)
    us = props(uL, uR, u)
    u, uL, uR = _combine_mm(us, satts[0], dg, W1, b1)
    us = props(uL, uR, u)
    res = _combine_out(us, satts[1], dg, Wp, bp)
    return res[:N, :OUT]
